# baseline (speedup 1.0000x reference)
import functools
import numpy as np
import jax
import jax.numpy as jnp
from jax import lax
from jax.experimental import pallas as pl
from jax.experimental.pallas import tpu as pltpu
from jax.experimental.pallas import tpu_sc as plsc

_INV_BN = np.float32(1.0) / np.sqrt(np.float32(1.0 + 1e-5))


def _flat_iota(shape):
    return (lax.broadcasted_iota(jnp.int32, shape, 0) * shape[1]
            + lax.broadcasted_iota(jnp.int32, shape, 1))


def _fps_call(pxg, pyg, pzg, pts_smem, K):
    R = pxg.shape[0]
    N = R * 128

    def body(px_ref, py_ref, pz_ref, ps_ref, o_ref):
        px = px_ref[...]
        py = py_ref[...]
        pz = pz_ref[...]
        fi = _flat_iota((R, 128))

        def dist_to(xj, yj, zj):
            dx = px - xj
            dy = py - yj
            dz = pz - zj
            return jnp.sqrt(dx * dx + dy * dy + dz * dz)

        x0 = ps_ref[0, 0]
        y0 = ps_ref[0, 1]
        z0 = ps_ref[0, 2]
        o_ref[0, 0] = x0
        o_ref[0, 1] = y0
        o_ref[0, 2] = z0

        def step(i, d):
            mx = jnp.max(d)
            j = jnp.min(jnp.where(d == mx, fi, jnp.int32(N)))
            xj = ps_ref[0, j * 3]
            yj = ps_ref[0, j * 3 + 1]
            zj = ps_ref[0, j * 3 + 2]
            o_ref[0, i * 3] = xj
            o_ref[0, i * 3 + 1] = yj
            o_ref[0, i * 3 + 2] = zj
            return jnp.minimum(d, dist_to(xj, yj, zj))

        lax.fori_loop(1, K, step, dist_to(x0, y0, z0))

    return pl.pallas_call(
        body,
        in_specs=[
            pl.BlockSpec((R, 128), lambda: (0, 0)),
            pl.BlockSpec((R, 128), lambda: (0, 0)),
            pl.BlockSpec((R, 128), lambda: (0, 0)),
            pl.BlockSpec((1, 3 * N), lambda: (0, 0),
                         memory_space=pltpu.SMEM),
        ],
        out_specs=pl.BlockSpec((1, 3 * K), lambda: (0, 0),
                               memory_space=pltpu.SMEM),
        out_shape=jax.ShapeDtypeStruct((1, 3 * K), jnp.float32),
    )(pxg, pyg, pzg, pts_smem)


def _sa_call(XinT, W1, b1c, w1x, w1y, w1z, W2, b2c, W3, b3c,
             centers, cxc, cyc, czc, pxr, pyr, pzr, radius, CB):
    Din, N = XinT.shape
    C = cxc.shape[0]
    H3 = W3.shape[0]
    r32 = np.float32(radius)
    ninf = np.float32(-np.inf)

    def body(x_ref, w1_ref, b1_ref, w1x_ref, w1y_ref, w1z_ref,
             w2_ref, b2_ref, w3_ref, b3_ref, c_ref,
             cx_ref, cy_ref, cz_ref, px_ref, py_ref, pz_ref, o_ref):
        baseT = jnp.dot(w1_ref[...], x_ref[...],
                        preferred_element_type=jnp.float32) + b1_ref[...]
        w1xv = w1x_ref[...]
        w1yv = w1y_ref[...]
        w1zv = w1z_ref[...]
        W2 = w2_ref[...].astype(jnp.bfloat16)
        b2v = b2_ref[...]
        W3 = w3_ref[...].astype(jnp.bfloat16)
        b3v = b3_ref[...]
        dx = cx_ref[...] - px_ref[...]
        dy = cy_ref[...] - py_ref[...]
        dz = cz_ref[...] - pz_ref[...]
        pen = jnp.where(
            jnp.sqrt(dx * dx + dy * dy + dz * dz) < r32, 0.0, ninf)
        li = lax.broadcasted_iota(jnp.int32, (H3, CB), 1)
        acc = jnp.zeros((H3, CB), jnp.float32)
        for c in range(CB):
            cxs = c_ref[c, 0]
            cys = c_ref[c, 1]
            czs = c_ref[c, 2]
            coffT = cxs * w1xv + cys * w1yv + czs * w1zv
            h = jnp.maximum(baseT - coffT, 0.0)
            h = jnp.maximum(
                jnp.dot(W2, h.astype(jnp.bfloat16),
                        preferred_element_type=jnp.float32) + b2v, 0.0)
            h = jnp.dot(W3, h.astype(jnp.bfloat16),
                        preferred_element_type=jnp.float32)
            m = jnp.max(h + pen[c:c + 1, :], axis=1, keepdims=True)
            acc = jnp.where(li == c, jnp.maximum(m + b3v, 0.0), acc)
        o_ref[0] = acc

    return pl.pallas_call(
        body,
        grid=(C // CB,),
        in_specs=[
            pl.BlockSpec((Din, N), lambda i: (0, 0)),
            pl.BlockSpec(W1.shape, lambda i: (0, 0)),
            pl.BlockSpec(b1c.shape, lambda i: (0, 0)),
            pl.BlockSpec(w1x.shape, lambda i: (0, 0)),
            pl.BlockSpec(w1y.shape, lambda i: (0, 0)),
            pl.BlockSpec(w1z.shape, lambda i: (0, 0)),
            pl.BlockSpec(W2.shape, lambda i: (0, 0)),
            pl.BlockSpec(b2c.shape, lambda i: (0, 0)),
            pl.BlockSpec(W3.shape, lambda i: (0, 0)),
            pl.BlockSpec(b3c.shape, lambda i: (0, 0)),
            pl.BlockSpec((CB, 3), lambda i: (i, 0),
                         memory_space=pltpu.SMEM),
            pl.BlockSpec((CB, 1), lambda i: (i, 0)),
            pl.BlockSpec((CB, 1), lambda i: (i, 0)),
            pl.BlockSpec((CB, 1), lambda i: (i, 0)),
            pl.BlockSpec((1, N), lambda i: (0, 0)),
            pl.BlockSpec((1, N), lambda i: (0, 0)),
            pl.BlockSpec((1, N), lambda i: (0, 0)),
        ],
        out_specs=pl.BlockSpec((1, H3, CB), lambda i: (i, 0, 0)),
        out_shape=jax.ShapeDtypeStruct((C // CB, H3, CB), jnp.float32),
    )(XinT, W1, b1c, w1x, w1y, w1z, W2, b2c, W3, b3c,
      centers, cxc, cyc, czc, pxr, pyr, pzr)


def _tail_call(c2cols, c2smem, f2, w1c, w1f, b1, w2, b2, w3, b3,
               wa, wb, bb, g0, be0, w2f, b2f, g1, be1):
    C2, F2 = f2.shape
    OUT = w2f.shape[1]

    def body(cx_ref, cy_ref, cz_ref, cs_ref, f_ref, w1c_ref, w1f_ref,
             b1_ref, w2_ref, b2_ref, w3_ref, b3_ref, wa_ref, wb_ref,
             bb_ref, g0_ref, be0_ref, w2f_ref, b2f_ref, g1_ref, be1_ref,
             o_ref):
        dx = cx_ref[...] - cs_ref[0, 0]
        dy = cy_ref[...] - cs_ref[0, 1]
        dz = cz_ref[...] - cs_ref[0, 2]
        W1c = w1c_ref[...]
        dpart = dx * W1c[0:1] + dy * W1c[1:2] + dz * W1c[2:3]
        f2v = f_ref[...]
        h = jnp.maximum(
            dpart
            + jnp.dot(f2v, w1f_ref[...], preferred_element_type=jnp.float32)
            + b1_ref[...], 0.0)
        h = jnp.maximum(
            jnp.dot(h, w2_ref[...], preferred_element_type=jnp.float32)
            + b2_ref[...], 0.0)
        h = jnp.maximum(
            jnp.dot(h, w3_ref[...], preferred_element_type=jnp.float32)
            + b3_ref[...], 0.0)
        fm = jnp.max(h, axis=0, keepdims=True)
        kmw = jnp.dot(fm, wb_ref[...], preferred_element_type=jnp.float32)
        y = (jnp.dot(f2v, wa_ref[...], preferred_element_type=jnp.float32)
             + kmw + bb_ref[...])
        y = jnp.maximum(g0_ref[...] * y * _INV_BN + be0_ref[...], 0.0)
        y = jnp.dot(y, w2f_ref[...], preferred_element_type=jnp.float32) \
            + b2f_ref[...]
        y = jnp.maximum(g1_ref[...] * y * _INV_BN + be1_ref[...], 0.0)
        o_ref[...] = y

    vspec = lambda a: pl.BlockSpec(a.shape, lambda: (0,) * a.ndim)
    args = (*c2cols, c2smem, f2, w1c, w1f, b1, w2, b2, w3, b3,
            wa, wb, bb, g0, be0, w2f, b2f, g1, be1)
    in_specs = [vspec(a) for a in args]
    in_specs[3] = pl.BlockSpec(c2smem.shape, lambda: (0, 0),
                               memory_space=pltpu.SMEM)
    return pl.pallas_call(
        body,
        in_specs=in_specs,
        out_specs=pl.BlockSpec((C2, OUT), lambda: (0, 0)),
        out_shape=jax.ShapeDtypeStruct((C2, OUT), jnp.float32),
    )(*args)


_NC, _NS, _L = 2, 16, 16
_NW = _NC * _NS


def _knn4_sc(qcoords, pcoords):
    Q = qcoords.shape[0]
    P = pcoords.shape[0]
    L = _L
    nq = Q // _NW
    ng = nq // L
    f32 = jnp.float32
    i32 = jnp.int32

    qx = qcoords[:, 0]
    qy = qcoords[:, 1]
    qz = qcoords[:, 2]
    px_s = jnp.repeat(pcoords[:, 0], L)
    py_s = jnp.repeat(pcoords[:, 1], L)
    pz_s = jnp.repeat(pcoords[:, 2], L)

    out_type = ([jax.ShapeDtypeStruct((Q,), f32) for _ in range(4)]
                + [jax.ShapeDtypeStruct((Q,), i32) for _ in range(4)])
    scratch = ([pltpu.VMEM((nq,), f32) for _ in range(3)]
               + [pltpu.VMEM((P * L,), f32) for _ in range(3)]
               + [pltpu.VMEM((nq,), f32) for _ in range(4)]
               + [pltpu.VMEM((nq,), i32) for _ in range(4)])

    mesh = plsc.VectorSubcoreMesh(core_axis_name="c", subcore_axis_name="s")

    @functools.partial(pl.kernel, mesh=mesh, out_type=out_type,
                       scratch_types=scratch)
    def knn_kernel(qx_h, qy_h, qz_h, px_h, py_h, pz_h,
                   sq0_h, sq1_h, sq2_h, sq3_h,
                   ik0_h, ik1_h, ik2_h, ik3_h,
                   qxv, qyv, qzv, pxv, pyv, pzv,
                   t0v, t1v, t2v, t3v, i0v, i1v, i2v, i3v):
        wid = lax.axis_index("s") * _NC + lax.axis_index("c")
        base = wid * nq
        pltpu.sync_copy(qx_h.at[pl.ds(base, nq)], qxv)
        pltpu.sync_copy(qy_h.at[pl.ds(base, nq)], qyv)
        pltpu.sync_copy(qz_h.at[pl.ds(base, nq)], qzv)
        pltpu.sync_copy(px_h, pxv)
        pltpu.sync_copy(py_h, pyv)
        pltpu.sync_copy(pz_h, pzv)
        for g in range(ng):
            qxg = qxv[pl.ds(g * L, L)]
            qyg = qyv[pl.ds(g * L, L)]
            qzg = qzv[pl.ds(g * L, L)]
            inf16 = jnp.full((L,), np.float32(np.inf), f32)
            zero16 = jnp.zeros((L,), i32)
            state = (inf16, inf16, inf16, inf16,
                     zero16, zero16, zero16, zero16)

            def point_body(p, st, qxg=qxg, qyg=qyg, qzg=qzg):
                t0, t1, t2, t3, i0, i1, i2, i3 = st
                pxs = pxv[pl.ds(p * L, L)]
                pys = pyv[pl.ds(p * L, L)]
                pzs = pzv[pl.ds(p * L, L)]
                dx = qxg - pxs
                dy = qyg - pys
                dz = qzg - pzs
                dv = dx * dx + dy * dy + dz * dz
                iv = jnp.full((L,), 0, i32) + p
                c0 = dv < t0
                t0n = jnp.where(c0, dv, t0)
                i0n = jnp.where(c0, iv, i0)
                dv1 = jnp.where(c0, t0, dv)
                iv1 = jnp.where(c0, i0, iv)
                c1 = dv1 < t1
                t1n = jnp.where(c1, dv1, t1)
                i1n = jnp.where(c1, iv1, i1)
                dv2 = jnp.where(c1, t1, dv1)
                iv2 = jnp.where(c1, i1, iv1)
                c2 = dv2 < t2
                t2n = jnp.where(c2, dv2, t2)
                i2n = jnp.where(c2, iv2, i2)
                dv3 = jnp.where(c2, t2, dv2)
                iv3 = jnp.where(c2, i2, iv2)
                c3 = dv3 < t3
                t3n = jnp.where(c3, dv3, t3)
                i3n = jnp.where(c3, iv3, i3)
                return (t0n, t1n, t2n, t3n, i0n, i1n, i2n, i3n)

            st = lax.fori_loop(0, P, point_body, state)
            t0v[pl.ds(g * L, L)] = st[0]
            t1v[pl.ds(g * L, L)] = st[1]
            t2v[pl.ds(g * L, L)] = st[2]
            t3v[pl.ds(g * L, L)] = st[3]
            i0v[pl.ds(g * L, L)] = st[4]
            i1v[pl.ds(g * L, L)] = st[5]
            i2v[pl.ds(g * L, L)] = st[6]
            i3v[pl.ds(g * L, L)] = st[7]
        pltpu.sync_copy(t0v, sq0_h.at[pl.ds(base, nq)])
        pltpu.sync_copy(t1v, sq1_h.at[pl.ds(base, nq)])
        pltpu.sync_copy(t2v, sq2_h.at[pl.ds(base, nq)])
        pltpu.sync_copy(t3v, sq3_h.at[pl.ds(base, nq)])
        pltpu.sync_copy(i0v, ik0_h.at[pl.ds(base, nq)])
        pltpu.sync_copy(i1v, ik1_h.at[pl.ds(base, nq)])
        pltpu.sync_copy(i2v, ik2_h.at[pl.ds(base, nq)])
        pltpu.sync_copy(i3v, ik3_h.at[pl.ds(base, nq)])

    res = knn_kernel(qx, qy, qz, px_s, py_s, pz_s)
    return list(res[:4]), list(res[4:])


def _fp_call(sqs, iks_in, from_f, f_prev, layer_arrays, bn_flags, QB):
    Q = sqs[0].shape[0]
    P, F = from_f.shape
    Dprev = f_prev.shape[1]
    OUT = layer_arrays[-1][0].shape[1]

    flat = []
    for arrs in layer_arrays:
        flat.extend(arrs)
    n_flat = len(flat)

    def body(*refs):
        s_refs = refs[0:4]
        i_refs = refs[4:8]
        ff_ref = refs[8]
        fp_ref = refs[9]
        lrefs = list(refs[10:10 + n_flat])
        o_ref = refs[10 + n_flat]
        cand = []
        for k in range(4):
            sq = s_refs[k][...]
            z = sq == 0.0
            d = jnp.where(z, 0.0, jnp.sqrt(jnp.where(z, 1.0, sq)))
            cand.append((d, i_refs[k][...]))

        def cswap(a, b):
            da, ia = a
            db, ib = b
            sw = (da > db) | ((da == db) & (ia > ib))
            lo = (jnp.where(sw, db, da), jnp.where(sw, ib, ia))
            hi = (jnp.where(sw, da, db), jnp.where(sw, ia, ib))
            return lo, hi

        for a, b in [(0, 1), (2, 3), (0, 2), (1, 3), (1, 2)]:
            cand[a], cand[b] = cswap(cand[a], cand[b])
        dks = [cand[k][0] for k in range(3)]
        iks = [cand[k][1] for k in range(3)]
        iz = [dk == 0.0 for dk in dks]
        any_zero = iz[0] | iz[1] | iz[2]
        raws = []
        for z, dk in zip(iz, dks):
            safe = jnp.where(z, 1.0, dk)
            raws.append(1.0 / (safe * safe))
        s = raws[0] + raws[1] + raws[2]
        col = lax.broadcasted_iota(jnp.int32, (QB, P), 1)
        Wc = jnp.zeros((QB, P), jnp.float32)
        for k in range(3):
            wk = jnp.where(any_zero, iz[k].astype(jnp.float32), raws[k] / s)
            Wc = Wc + jnp.where(col == iks[k], wk, 0.0)
        km = jnp.dot(Wc, ff_ref[...], preferred_element_type=jnp.float32)
        wp = lrefs[0][...]
        wk_ = lrefs[1][...]
        b = lrefs[2][...]
        x = (jnp.dot(fp_ref[...], wp, preferred_element_type=jnp.float32)
             + jnp.dot(km, wk_, preferred_element_type=jnp.float32) + b)
        li = 3
        if bn_flags[0]:
            x = jnp.maximum(lrefs[li][...] * x * _INV_BN
                            + lrefs[li + 1][...], 0.0)
            li += 2
        for has_bn in bn_flags[1:]:
            w = lrefs[li][...]
            b = lrefs[li + 1][...]
            x = jnp.dot(x, w, preferred_element_type=jnp.float32) + b
            li += 2
            if has_bn:
                x = jnp.maximum(lrefs[li][...] * x * _INV_BN
                                + lrefs[li + 1][...], 0.0)
                li += 2
        o_ref[...] = x

    def full2(a):
        s = a.shape
        return pl.BlockSpec(s, lambda i: (0, 0))

    qspec = pl.BlockSpec((QB, 1), lambda i: (i, 0))
    in_specs = ([qspec] * 8 + [
        full2(from_f),
        pl.BlockSpec((QB, Dprev), lambda i: (i, 0)),
    ] + [full2(a) for a in flat])
    return pl.pallas_call(
        body,
        grid=(Q // QB,),
        in_specs=in_specs,
        out_specs=pl.BlockSpec((QB, OUT), lambda i: (i, 0)),
        out_shape=jax.ShapeDtypeStruct((Q, OUT), jnp.float32),
    )(*sqs, *iks_in, from_f, f_prev, *flat)


def kernel(coords, features, params):
    coords = coords.astype(jnp.float32)
    features = features.astype(jnp.float32)
    N = coords.shape[0]

    pxg = coords[:, 0].reshape(N // 128, 128)
    pyg = coords[:, 1].reshape(N // 128, 128)
    pzg = coords[:, 2].reshape(N // 128, 128)

    c1coords = _fps_call(pxg, pyg, pzg, coords.reshape(1, -1),
                         512).reshape(512, 3)
    c1x = c1coords[:, 0].reshape(4, 128)
    c1y = c1coords[:, 1].reshape(4, 128)
    c1z = c1coords[:, 2].reshape(4, 128)
    c2coords = _fps_call(c1x, c1y, c1z, c1coords.reshape(1, -1),
                         128).reshape(128, 3)
    c2x = c2coords[:, 0].reshape(1, 128)
    c2y = c2coords[:, 1].reshape(1, 128)
    c2z = c2coords[:, 2].reshape(1, 128)

    def _unblock(o):
        return jnp.transpose(o, (1, 0, 2)).reshape(o.shape[1], -1)

    sa0 = params['sa'][0]
    XinT1 = jnp.concatenate([coords.T, features.T], axis=0)
    f1T = _sa_call(
        XinT1, sa0['W1'], sa0['b1'][:, None],
        sa0['W1'][:, 0:1], sa0['W1'][:, 1:2], sa0['W1'][:, 2:3],
        sa0['W2'], sa0['b2'][:, None], sa0['W3'], sa0['b3'][:, None],
        c1coords,
        c1x.reshape(512, 1), c1y.reshape(512, 1), c1z.reshape(512, 1),
        coords[:, 0].reshape(1, N), coords[:, 1].reshape(1, N),
        coords[:, 2].reshape(1, N),
        0.2, CB=8)
    f1T = _unblock(f1T)
    f1 = f1T.T

    sa1 = params['sa'][1]
    c1coordsT = jnp.stack(
        [c1x.reshape(-1), c1y.reshape(-1), c1z.reshape(-1)], axis=0)
    XinT2 = jnp.concatenate([c1coordsT, f1T], axis=0)
    f2T = _sa_call(
        XinT2, sa1['W1'], sa1['b1'][:, None],
        sa1['W1'][:, 0:1], sa1['W1'][:, 1:2], sa1['W1'][:, 2:3],
        sa1['W2'], sa1['b2'][:, None], sa1['W3'], sa1['b3'][:, None],
        c2coords,
        c2x.reshape(128, 1), c2y.reshape(128, 1), c2z.reshape(128, 1),
        c1x.reshape(1, 512), c1y.reshape(1, 512), c1z.reshape(1, 512),
        0.4, CB=8)
    f2 = _unblock(f2T).T

    sa2 = params['sa'][2]
    fp0 = params['fp'][0]
    W1t = sa2['W1'].T
    fp0W0t = fp0[0]['W'].T
    g2 = _tail_call(
        (c2x.reshape(128, 1), c2y.reshape(128, 1), c2z.reshape(128, 1)),
        c2coords, f2,
        W1t[:3], W1t[3:], sa2['b1'][None, :],
        sa2['W2'].T, sa2['b2'][None, :], sa2['W3'].T, sa2['b3'][None, :],
        fp0W0t[:256], fp0W0t[256:], fp0[0]['b'][None, :],
        fp0[0]['gamma'][None, :], fp0[0]['beta'][None, :],
        fp0[1]['W'].T, fp0[1]['b'][None, :],
        fp0[1]['gamma'][None, :], fp0[1]['beta'][None, :])

    fp1 = params['fp'][1]
    W0t = fp1[0]['W'].T
    layer_arrays1 = [
        (W0t[:128], W0t[128:], fp1[0]['b'][None, :],
         fp1[0]['gamma'][None, :], fp1[0]['beta'][None, :]),
        (fp1[1]['W'].T, fp1[1]['b'][None, :],
         fp1[1]['gamma'][None, :], fp1[1]['beta'][None, :]),
    ]
    sq1s, ik1s = _knn4_sc(c1coords, c2coords)
    g1 = _fp_call(
        [a.reshape(512, 1) for a in sq1s],
        [a.reshape(512, 1) for a in ik1s],
        g2, f1, layer_arrays1, [True, True], QB=512)

    fp2 = params['fp'][2]
    W0t2 = fp2[0]['W'].T
    layer_arrays2 = [
        (W0t2[:3], W0t2[3:], fp2[0]['b'][None, :],
         fp2[0]['gamma'][None, :], fp2[0]['beta'][None, :]),
        (fp2[1]['W'].T, fp2[1]['b'][None, :],
         fp2[1]['gamma'][None, :], fp2[1]['beta'][None, :]),
        (fp2[2]['W'].T, fp2[2]['b'][None, :]),
    ]
    sq2s, ik2s = _knn4_sc(coords, c1coords)
    out = _fp_call(
        [a.reshape(4096, 1) for a in sq2s],
        [a.reshape(4096, 1) for a in ik2s],
        g1, features, layer_arrays2, [True, True, False], QB=512)
    return out


# speedup vs baseline: 1.0004x; 1.0004x over previous
import functools
import numpy as np
import jax
import jax.numpy as jnp
from jax import lax
from jax.experimental import pallas as pl
from jax.experimental.pallas import tpu as pltpu
from jax.experimental.pallas import tpu_sc as plsc

_INV_BN = np.float32(1.0) / np.sqrt(np.float32(1.0 + 1e-5))


def _flat_iota(shape):
    return (lax.broadcasted_iota(jnp.int32, shape, 0) * shape[1]
            + lax.broadcasted_iota(jnp.int32, shape, 1))


def _fps_call(pxg, pyg, pzg, pts_smem, K):
    R = pxg.shape[0]
    N = R * 128

    def body(px_ref, py_ref, pz_ref, ps_ref, o_ref):
        px = px_ref[...]
        py = py_ref[...]
        pz = pz_ref[...]
        fi = _flat_iota((R, 128))

        def dist_to(xj, yj, zj):
            dx = px - xj
            dy = py - yj
            dz = pz - zj
            return jnp.sqrt(dx * dx + dy * dy + dz * dz)

        x0 = ps_ref[0, 0]
        y0 = ps_ref[0, 1]
        z0 = ps_ref[0, 2]
        o_ref[0, 0] = x0
        o_ref[0, 1] = y0
        o_ref[0, 2] = z0

        def step(i, d):
            mx = jnp.max(d)
            j = jnp.min(jnp.where(d == mx, fi, jnp.int32(N)))
            xj = ps_ref[0, j * 3]
            yj = ps_ref[0, j * 3 + 1]
            zj = ps_ref[0, j * 3 + 2]
            o_ref[0, i * 3] = xj
            o_ref[0, i * 3 + 1] = yj
            o_ref[0, i * 3 + 2] = zj
            return jnp.minimum(d, dist_to(xj, yj, zj))

        lax.fori_loop(1, K, step, dist_to(x0, y0, z0))

    return pl.pallas_call(
        body,
        in_specs=[
            pl.BlockSpec((R, 128), lambda: (0, 0)),
            pl.BlockSpec((R, 128), lambda: (0, 0)),
            pl.BlockSpec((R, 128), lambda: (0, 0)),
            pl.BlockSpec((1, 3 * N), lambda: (0, 0),
                         memory_space=pltpu.SMEM),
        ],
        out_specs=pl.BlockSpec((1, 3 * K), lambda: (0, 0),
                               memory_space=pltpu.SMEM),
        out_shape=jax.ShapeDtypeStruct((1, 3 * K), jnp.float32),
    )(pxg, pyg, pzg, pts_smem)


def _sa_call(XinT, W1, b1c, w1x, w1y, w1z, W2, b2c, W3, b3c,
             centers, cxc, cyc, czc, pxr, pyr, pzr, radius, CB):
    Din, N = XinT.shape
    C = cxc.shape[0]
    H3 = W3.shape[0]
    r32 = np.float32(radius)
    ninf = np.float32(-np.inf)

    def body(x_ref, w1_ref, b1_ref, w1x_ref, w1y_ref, w1z_ref,
             w2_ref, b2_ref, w3_ref, b3_ref, c_ref,
             cx_ref, cy_ref, cz_ref, px_ref, py_ref, pz_ref, o_ref):
        baseT = jnp.dot(w1_ref[...], x_ref[...],
                        preferred_element_type=jnp.float32) + b1_ref[...]
        w1xv = w1x_ref[...]
        w1yv = w1y_ref[...]
        w1zv = w1z_ref[...]
        W2 = w2_ref[...].astype(jnp.bfloat16)
        b2v = b2_ref[...]
        W3 = w3_ref[...].astype(jnp.bfloat16)
        b3v = b3_ref[...]
        dx = cx_ref[...] - px_ref[...]
        dy = cy_ref[...] - py_ref[...]
        dz = cz_ref[...] - pz_ref[...]
        pen = jnp.where(
            jnp.sqrt(dx * dx + dy * dy + dz * dz) < r32, 0.0, ninf)
        li = lax.broadcasted_iota(jnp.int32, (H3, CB), 1)
        acc = jnp.zeros((H3, CB), jnp.float32)
        for c in range(CB):
            cxs = c_ref[c, 0]
            cys = c_ref[c, 1]
            czs = c_ref[c, 2]
            coffT = cxs * w1xv + cys * w1yv + czs * w1zv
            h = jnp.maximum(baseT - coffT, 0.0)
            h = jnp.maximum(
                jnp.dot(W2, h.astype(jnp.bfloat16),
                        preferred_element_type=jnp.float32) + b2v, 0.0)
            h = jnp.dot(W3, h.astype(jnp.bfloat16),
                        preferred_element_type=jnp.float32)
            m = jnp.max(h + pen[c:c + 1, :], axis=1, keepdims=True)
            acc = jnp.where(li == c, jnp.maximum(m + b3v, 0.0), acc)
        o_ref[0] = acc

    return pl.pallas_call(
        body,
        grid=(C // CB,),
        in_specs=[
            pl.BlockSpec((Din, N), lambda i: (0, 0)),
            pl.BlockSpec(W1.shape, lambda i: (0, 0)),
            pl.BlockSpec(b1c.shape, lambda i: (0, 0)),
            pl.BlockSpec(w1x.shape, lambda i: (0, 0)),
            pl.BlockSpec(w1y.shape, lambda i: (0, 0)),
            pl.BlockSpec(w1z.shape, lambda i: (0, 0)),
            pl.BlockSpec(W2.shape, lambda i: (0, 0)),
            pl.BlockSpec(b2c.shape, lambda i: (0, 0)),
            pl.BlockSpec(W3.shape, lambda i: (0, 0)),
            pl.BlockSpec(b3c.shape, lambda i: (0, 0)),
            pl.BlockSpec((CB, 3), lambda i: (i, 0),
                         memory_space=pltpu.SMEM),
            pl.BlockSpec((CB, 1), lambda i: (i, 0)),
            pl.BlockSpec((CB, 1), lambda i: (i, 0)),
            pl.BlockSpec((CB, 1), lambda i: (i, 0)),
            pl.BlockSpec((1, N), lambda i: (0, 0)),
            pl.BlockSpec((1, N), lambda i: (0, 0)),
            pl.BlockSpec((1, N), lambda i: (0, 0)),
        ],
        out_specs=pl.BlockSpec((1, H3, CB), lambda i: (i, 0, 0)),
        out_shape=jax.ShapeDtypeStruct((C // CB, H3, CB), jnp.float32),
    )(XinT, W1, b1c, w1x, w1y, w1z, W2, b2c, W3, b3c,
      centers, cxc, cyc, czc, pxr, pyr, pzr)


def _tail_call(c2cols, c2smem, f2, w1c, w1f, b1, w2, b2, w3, b3,
               wa, wb, bb, g0, be0, w2f, b2f, g1, be1):
    C2, F2 = f2.shape
    OUT = w2f.shape[1]

    def body(cx_ref, cy_ref, cz_ref, cs_ref, f_ref, w1c_ref, w1f_ref,
             b1_ref, w2_ref, b2_ref, w3_ref, b3_ref, wa_ref, wb_ref,
             bb_ref, g0_ref, be0_ref, w2f_ref, b2f_ref, g1_ref, be1_ref,
             o_ref):
        dx = cx_ref[...] - cs_ref[0, 0]
        dy = cy_ref[...] - cs_ref[0, 1]
        dz = cz_ref[...] - cs_ref[0, 2]
        W1c = w1c_ref[...]
        dpart = dx * W1c[0:1] + dy * W1c[1:2] + dz * W1c[2:3]
        f2v = f_ref[...]
        h = jnp.maximum(
            dpart
            + jnp.dot(f2v, w1f_ref[...], preferred_element_type=jnp.float32)
            + b1_ref[...], 0.0)
        h = jnp.maximum(
            jnp.dot(h, w2_ref[...], preferred_element_type=jnp.float32)
            + b2_ref[...], 0.0)
        h = jnp.maximum(
            jnp.dot(h, w3_ref[...], preferred_element_type=jnp.float32)
            + b3_ref[...], 0.0)
        fm = jnp.max(h, axis=0, keepdims=True)
        kmw = jnp.dot(fm, wb_ref[...], preferred_element_type=jnp.float32)
        y = (jnp.dot(f2v, wa_ref[...], preferred_element_type=jnp.float32)
             + kmw + bb_ref[...])
        y = jnp.maximum(g0_ref[...] * y * _INV_BN + be0_ref[...], 0.0)
        y = jnp.dot(y, w2f_ref[...], preferred_element_type=jnp.float32) \
            + b2f_ref[...]
        y = jnp.maximum(g1_ref[...] * y * _INV_BN + be1_ref[...], 0.0)
        o_ref[...] = y

    vspec = lambda a: pl.BlockSpec(a.shape, lambda: (0,) * a.ndim)
    args = (*c2cols, c2smem, f2, w1c, w1f, b1, w2, b2, w3, b3,
            wa, wb, bb, g0, be0, w2f, b2f, g1, be1)
    in_specs = [vspec(a) for a in args]
    in_specs[3] = pl.BlockSpec(c2smem.shape, lambda: (0, 0),
                               memory_space=pltpu.SMEM)
    return pl.pallas_call(
        body,
        in_specs=in_specs,
        out_specs=pl.BlockSpec((C2, OUT), lambda: (0, 0)),
        out_shape=jax.ShapeDtypeStruct((C2, OUT), jnp.float32),
    )(*args)


_NC, _NS, _L = 2, 16, 16
_NW = _NC * _NS


def _knn4_sc(qcoords, pcoords):
    Q = qcoords.shape[0]
    P = pcoords.shape[0]
    L = _L
    nq = Q // _NW
    ng = nq // L
    f32 = jnp.float32
    i32 = jnp.int32

    qx = qcoords[:, 0]
    qy = qcoords[:, 1]
    qz = qcoords[:, 2]
    px_s = jnp.repeat(pcoords[:, 0], L)
    py_s = jnp.repeat(pcoords[:, 1], L)
    pz_s = jnp.repeat(pcoords[:, 2], L)

    out_type = ([jax.ShapeDtypeStruct((Q,), f32) for _ in range(4)]
                + [jax.ShapeDtypeStruct((Q,), i32) for _ in range(4)])
    scratch = ([pltpu.VMEM((nq,), f32) for _ in range(3)]
               + [pltpu.VMEM((P * L,), f32) for _ in range(3)]
               + [pltpu.VMEM((nq,), f32) for _ in range(4)]
               + [pltpu.VMEM((nq,), i32) for _ in range(4)])

    mesh = plsc.VectorSubcoreMesh(core_axis_name="c", subcore_axis_name="s")

    @functools.partial(pl.kernel, mesh=mesh, out_type=out_type,
                       scratch_types=scratch)
    def knn_kernel(qx_h, qy_h, qz_h, px_h, py_h, pz_h,
                   sq0_h, sq1_h, sq2_h, sq3_h,
                   ik0_h, ik1_h, ik2_h, ik3_h,
                   qxv, qyv, qzv, pxv, pyv, pzv,
                   t0v, t1v, t2v, t3v, i0v, i1v, i2v, i3v):
        wid = lax.axis_index("s") * _NC + lax.axis_index("c")
        base = wid * nq
        pltpu.sync_copy(qx_h.at[pl.ds(base, nq)], qxv)
        pltpu.sync_copy(qy_h.at[pl.ds(base, nq)], qyv)
        pltpu.sync_copy(qz_h.at[pl.ds(base, nq)], qzv)
        pltpu.sync_copy(px_h, pxv)
        pltpu.sync_copy(py_h, pyv)
        pltpu.sync_copy(pz_h, pzv)
        for g in range(ng):
            qxg = qxv[pl.ds(g * L, L)]
            qyg = qyv[pl.ds(g * L, L)]
            qzg = qzv[pl.ds(g * L, L)]
            inf16 = jnp.full((L,), np.float32(np.inf), f32)
            zero16 = jnp.zeros((L,), i32)
            state = (inf16, inf16, inf16, inf16,
                     zero16, zero16, zero16, zero16)

            def point_body(p, st, qxg=qxg, qyg=qyg, qzg=qzg):
                t0, t1, t2, t3, i0, i1, i2, i3 = st
                pxs = pxv[pl.ds(p * L, L)]
                pys = pyv[pl.ds(p * L, L)]
                pzs = pzv[pl.ds(p * L, L)]
                dx = qxg - pxs
                dy = qyg - pys
                dz = qzg - pzs
                dv = dx * dx + dy * dy + dz * dz
                iv = jnp.full((L,), 0, i32) + p
                c0 = dv < t0
                t0n = jnp.where(c0, dv, t0)
                i0n = jnp.where(c0, iv, i0)
                dv1 = jnp.where(c0, t0, dv)
                iv1 = jnp.where(c0, i0, iv)
                c1 = dv1 < t1
                t1n = jnp.where(c1, dv1, t1)
                i1n = jnp.where(c1, iv1, i1)
                dv2 = jnp.where(c1, t1, dv1)
                iv2 = jnp.where(c1, i1, iv1)
                c2 = dv2 < t2
                t2n = jnp.where(c2, dv2, t2)
                i2n = jnp.where(c2, iv2, i2)
                dv3 = jnp.where(c2, t2, dv2)
                iv3 = jnp.where(c2, i2, iv2)
                c3 = dv3 < t3
                t3n = jnp.where(c3, dv3, t3)
                i3n = jnp.where(c3, iv3, i3)
                return (t0n, t1n, t2n, t3n, i0n, i1n, i2n, i3n)

            st = lax.fori_loop(0, P, point_body, state, unroll=4)
            t0v[pl.ds(g * L, L)] = st[0]
            t1v[pl.ds(g * L, L)] = st[1]
            t2v[pl.ds(g * L, L)] = st[2]
            t3v[pl.ds(g * L, L)] = st[3]
            i0v[pl.ds(g * L, L)] = st[4]
            i1v[pl.ds(g * L, L)] = st[5]
            i2v[pl.ds(g * L, L)] = st[6]
            i3v[pl.ds(g * L, L)] = st[7]
        pltpu.sync_copy(t0v, sq0_h.at[pl.ds(base, nq)])
        pltpu.sync_copy(t1v, sq1_h.at[pl.ds(base, nq)])
        pltpu.sync_copy(t2v, sq2_h.at[pl.ds(base, nq)])
        pltpu.sync_copy(t3v, sq3_h.at[pl.ds(base, nq)])
        pltpu.sync_copy(i0v, ik0_h.at[pl.ds(base, nq)])
        pltpu.sync_copy(i1v, ik1_h.at[pl.ds(base, nq)])
        pltpu.sync_copy(i2v, ik2_h.at[pl.ds(base, nq)])
        pltpu.sync_copy(i3v, ik3_h.at[pl.ds(base, nq)])

    res = knn_kernel(qx, qy, qz, px_s, py_s, pz_s)
    return list(res[:4]), list(res[4:])


def _fp_call(sqs, iks_in, from_f, f_prev, layer_arrays, bn_flags, QB):
    Q = sqs[0].shape[0]
    P, F = from_f.shape
    Dprev = f_prev.shape[1]
    OUT = layer_arrays[-1][0].shape[1]

    flat = []
    for arrs in layer_arrays:
        flat.extend(arrs)
    n_flat = len(flat)

    def body(*refs):
        s_refs = refs[0:4]
        i_refs = refs[4:8]
        ff_ref = refs[8]
        fp_ref = refs[9]
        lrefs = list(refs[10:10 + n_flat])
        o_ref = refs[10 + n_flat]
        cand = []
        for k in range(4):
            sq = s_refs[k][...]
            z = sq == 0.0
            d = jnp.where(z, 0.0, jnp.sqrt(jnp.where(z, 1.0, sq)))
            cand.append((d, i_refs[k][...]))

        def cswap(a, b):
            da, ia = a
            db, ib = b
            sw = (da > db) | ((da == db) & (ia > ib))
            lo = (jnp.where(sw, db, da), jnp.where(sw, ib, ia))
            hi = (jnp.where(sw, da, db), jnp.where(sw, ia, ib))
            return lo, hi

        for a, b in [(0, 1), (2, 3), (0, 2), (1, 3), (1, 2)]:
            cand[a], cand[b] = cswap(cand[a], cand[b])
        dks = [cand[k][0] for k in range(3)]
        iks = [cand[k][1] for k in range(3)]
        iz = [dk == 0.0 for dk in dks]
        any_zero = iz[0] | iz[1] | iz[2]
        raws = []
        for z, dk in zip(iz, dks):
            safe = jnp.where(z, 1.0, dk)
            raws.append(1.0 / (safe * safe))
        s = raws[0] + raws[1] + raws[2]
        col = lax.broadcasted_iota(jnp.int32, (QB, P), 1)
        Wc = jnp.zeros((QB, P), jnp.float32)
        for k in range(3):
            wk = jnp.where(any_zero, iz[k].astype(jnp.float32), raws[k] / s)
            Wc = Wc + jnp.where(col == iks[k], wk, 0.0)
        km = jnp.dot(Wc, ff_ref[...], preferred_element_type=jnp.float32)
        wp = lrefs[0][...]
        wk_ = lrefs[1][...]
        b = lrefs[2][...]
        x = (jnp.dot(fp_ref[...], wp, preferred_element_type=jnp.float32)
             + jnp.dot(km, wk_, preferred_element_type=jnp.float32) + b)
        li = 3
        if bn_flags[0]:
            x = jnp.maximum(lrefs[li][...] * x * _INV_BN
                            + lrefs[li + 1][...], 0.0)
            li += 2
        for has_bn in bn_flags[1:]:
            w = lrefs[li][...]
            b = lrefs[li + 1][...]
            x = jnp.dot(x, w, preferred_element_type=jnp.float32) + b
            li += 2
            if has_bn:
                x = jnp.maximum(lrefs[li][...] * x * _INV_BN
                                + lrefs[li + 1][...], 0.0)
                li += 2
        o_ref[...] = x

    def full2(a):
        s = a.shape
        return pl.BlockSpec(s, lambda i: (0, 0))

    qspec = pl.BlockSpec((QB, 1), lambda i: (i, 0))
    in_specs = ([qspec] * 8 + [
        full2(from_f),
        pl.BlockSpec((QB, Dprev), lambda i: (i, 0)),
    ] + [full2(a) for a in flat])
    return pl.pallas_call(
        body,
        grid=(Q // QB,),
        in_specs=in_specs,
        out_specs=pl.BlockSpec((QB, OUT), lambda i: (i, 0)),
        out_shape=jax.ShapeDtypeStruct((Q, OUT), jnp.float32),
    )(*sqs, *iks_in, from_f, f_prev, *flat)


def kernel(coords, features, params):
    coords = coords.astype(jnp.float32)
    features = features.astype(jnp.float32)
    N = coords.shape[0]

    pxg = coords[:, 0].reshape(N // 128, 128)
    pyg = coords[:, 1].reshape(N // 128, 128)
    pzg = coords[:, 2].reshape(N // 128, 128)

    c1coords = _fps_call(pxg, pyg, pzg, coords.reshape(1, -1),
                         512).reshape(512, 3)
    c1x = c1coords[:, 0].reshape(4, 128)
    c1y = c1coords[:, 1].reshape(4, 128)
    c1z = c1coords[:, 2].reshape(4, 128)
    c2coords = _fps_call(c1x, c1y, c1z, c1coords.reshape(1, -1),
                         128).reshape(128, 3)
    c2x = c2coords[:, 0].reshape(1, 128)
    c2y = c2coords[:, 1].reshape(1, 128)
    c2z = c2coords[:, 2].reshape(1, 128)

    sq2s, ik2s = _knn4_sc(coords, c1coords)
    sq1s, ik1s = _knn4_sc(c1coords, c2coords)

    def _unblock(o):
        return jnp.transpose(o, (1, 0, 2)).reshape(o.shape[1], -1)

    sa0 = params['sa'][0]
    XinT1 = jnp.concatenate([coords.T, features.T], axis=0)
    f1T = _sa_call(
        XinT1, sa0['W1'], sa0['b1'][:, None],
        sa0['W1'][:, 0:1], sa0['W1'][:, 1:2], sa0['W1'][:, 2:3],
        sa0['W2'], sa0['b2'][:, None], sa0['W3'], sa0['b3'][:, None],
        c1coords,
        c1x.reshape(512, 1), c1y.reshape(512, 1), c1z.reshape(512, 1),
        coords[:, 0].reshape(1, N), coords[:, 1].reshape(1, N),
        coords[:, 2].reshape(1, N),
        0.2, CB=8)
    f1T = _unblock(f1T)
    f1 = f1T.T

    sa1 = params['sa'][1]
    c1coordsT = jnp.stack(
        [c1x.reshape(-1), c1y.reshape(-1), c1z.reshape(-1)], axis=0)
    XinT2 = jnp.concatenate([c1coordsT, f1T], axis=0)
    f2T = _sa_call(
        XinT2, sa1['W1'], sa1['b1'][:, None],
        sa1['W1'][:, 0:1], sa1['W1'][:, 1:2], sa1['W1'][:, 2:3],
        sa1['W2'], sa1['b2'][:, None], sa1['W3'], sa1['b3'][:, None],
        c2coords,
        c2x.reshape(128, 1), c2y.reshape(128, 1), c2z.reshape(128, 1),
        c1x.reshape(1, 512), c1y.reshape(1, 512), c1z.reshape(1, 512),
        0.4, CB=8)
    f2 = _unblock(f2T).T

    sa2 = params['sa'][2]
    fp0 = params['fp'][0]
    W1t = sa2['W1'].T
    fp0W0t = fp0[0]['W'].T
    g2 = _tail_call(
        (c2x.reshape(128, 1), c2y.reshape(128, 1), c2z.reshape(128, 1)),
        c2coords, f2,
        W1t[:3], W1t[3:], sa2['b1'][None, :],
        sa2['W2'].T, sa2['b2'][None, :], sa2['W3'].T, sa2['b3'][None, :],
        fp0W0t[:256], fp0W0t[256:], fp0[0]['b'][None, :],
        fp0[0]['gamma'][None, :], fp0[0]['beta'][None, :],
        fp0[1]['W'].T, fp0[1]['b'][None, :],
        fp0[1]['gamma'][None, :], fp0[1]['beta'][None, :])

    fp1 = params['fp'][1]
    W0t = fp1[0]['W'].T
    layer_arrays1 = [
        (W0t[:128], W0t[128:], fp1[0]['b'][None, :],
         fp1[0]['gamma'][None, :], fp1[0]['beta'][None, :]),
        (fp1[1]['W'].T, fp1[1]['b'][None, :],
         fp1[1]['gamma'][None, :], fp1[1]['beta'][None, :]),
    ]
    g1 = _fp_call(
        [a.reshape(512, 1) for a in sq1s],
        [a.reshape(512, 1) for a in ik1s],
        g2, f1, layer_arrays1, [True, True], QB=512)

    fp2 = params['fp'][2]
    W0t2 = fp2[0]['W'].T
    layer_arrays2 = [
        (W0t2[:3], W0t2[3:], fp2[0]['b'][None, :],
         fp2[0]['gamma'][None, :], fp2[0]['beta'][None, :]),
        (fp2[1]['W'].T, fp2[1]['b'][None, :],
         fp2[1]['gamma'][None, :], fp2[1]['beta'][None, :]),
        (fp2[2]['W'].T, fp2[2]['b'][None, :]),
    ]
    out = _fp_call(
        [a.reshape(4096, 1) for a in sq2s],
        [a.reshape(4096, 1) for a in ik2s],
        g1, features, layer_arrays2, [True, True, False], QB=512)
    return out


# speedup vs baseline: 1.0248x; 1.0243x over previous
import functools
import numpy as np
import jax
import jax.numpy as jnp
from jax import lax
from jax.experimental import pallas as pl
from jax.experimental.pallas import tpu as pltpu
from jax.experimental.pallas import tpu_sc as plsc

_INV_BN = np.float32(1.0) / np.sqrt(np.float32(1.0 + 1e-5))


def _flat_iota(shape):
    return (lax.broadcasted_iota(jnp.int32, shape, 0) * shape[1]
            + lax.broadcasted_iota(jnp.int32, shape, 1))


def _fps_call(pxg, pyg, pzg, pts_smem, K):
    R = pxg.shape[0]
    N = R * 128

    def body(px_ref, py_ref, pz_ref, ps_ref, o_ref):
        px = px_ref[...]
        py = py_ref[...]
        pz = pz_ref[...]
        fi = _flat_iota((R, 128))

        def dist_to(xj, yj, zj):
            dx = px - xj
            dy = py - yj
            dz = pz - zj
            return jnp.sqrt(dx * dx + dy * dy + dz * dz)

        x0 = ps_ref[0, 0]
        y0 = ps_ref[0, 1]
        z0 = ps_ref[0, 2]
        o_ref[0, 0] = x0
        o_ref[0, 1] = y0
        o_ref[0, 2] = z0

        def step(i, d):
            mx = jnp.max(d)
            j = jnp.min(jnp.where(d == mx, fi, jnp.int32(N)))
            xj = ps_ref[0, j * 3]
            yj = ps_ref[0, j * 3 + 1]
            zj = ps_ref[0, j * 3 + 2]
            o_ref[0, i * 3] = xj
            o_ref[0, i * 3 + 1] = yj
            o_ref[0, i * 3 + 2] = zj
            return jnp.minimum(d, dist_to(xj, yj, zj))

        lax.fori_loop(1, K, step, dist_to(x0, y0, z0))

    return pl.pallas_call(
        body,
        in_specs=[
            pl.BlockSpec((R, 128), lambda: (0, 0)),
            pl.BlockSpec((R, 128), lambda: (0, 0)),
            pl.BlockSpec((R, 128), lambda: (0, 0)),
            pl.BlockSpec((1, 3 * N), lambda: (0, 0),
                         memory_space=pltpu.SMEM),
        ],
        out_specs=pl.BlockSpec((1, 3 * K), lambda: (0, 0),
                               memory_space=pltpu.SMEM),
        out_shape=jax.ShapeDtypeStruct((1, 3 * K), jnp.float32),
    )(pxg, pyg, pzg, pts_smem)


def _sa_call(XinT, W1, b1c, w1x, w1y, w1z, W2, b2c, W3, b3c,
             centers, cxc, cyc, czc, pxr, pyr, pzr, radius, CB):
    Din, N = XinT.shape
    C = cxc.shape[0]
    H3 = W3.shape[0]
    r32 = np.float32(radius)
    ninf = np.float32(-np.inf)

    def body(x_ref, w1_ref, b1_ref, w1x_ref, w1y_ref, w1z_ref,
             w2_ref, b2_ref, w3_ref, b3_ref, c_ref,
             cx_ref, cy_ref, cz_ref, px_ref, py_ref, pz_ref, o_ref):
        bf = jnp.bfloat16
        baseT = (jnp.dot(w1_ref[...], x_ref[...],
                         preferred_element_type=jnp.float32)
                 + b1_ref[...]).astype(bf)
        w1xv = w1x_ref[...]
        w1yv = w1y_ref[...]
        w1zv = w1z_ref[...]
        W2 = w2_ref[...].astype(bf)
        b2v = b2_ref[...].astype(bf)
        W3 = w3_ref[...].astype(bf)
        b3v = b3_ref[...]
        dx = cx_ref[...] - px_ref[...]
        dy = cy_ref[...] - py_ref[...]
        dz = cz_ref[...] - pz_ref[...]
        pen = jnp.where(
            jnp.sqrt(dx * dx + dy * dy + dz * dz) < r32, 0.0, ninf).astype(bf)
        li = lax.broadcasted_iota(jnp.int32, (H3, CB), 1)
        acc = jnp.zeros((H3, CB), jnp.float32)
        for c in range(CB):
            cxs = c_ref[c, 0]
            cys = c_ref[c, 1]
            czs = c_ref[c, 2]
            coffT = (cxs * w1xv + cys * w1yv + czs * w1zv).astype(bf)
            h = jnp.maximum(baseT - coffT, 0.0)
            h = jnp.maximum(
                jnp.dot(W2, h,
                        preferred_element_type=jnp.float32).astype(bf)
                + b2v, 0.0)
            h = jnp.dot(W3, h, preferred_element_type=jnp.float32).astype(bf)
            m = jnp.max(h + pen[c:c + 1, :], axis=1, keepdims=True)
            acc = jnp.where(li == c,
                            jnp.maximum(m.astype(jnp.float32) + b3v, 0.0),
                            acc)
        o_ref[0] = acc

    return pl.pallas_call(
        body,
        grid=(C // CB,),
        in_specs=[
            pl.BlockSpec((Din, N), lambda i: (0, 0)),
            pl.BlockSpec(W1.shape, lambda i: (0, 0)),
            pl.BlockSpec(b1c.shape, lambda i: (0, 0)),
            pl.BlockSpec(w1x.shape, lambda i: (0, 0)),
            pl.BlockSpec(w1y.shape, lambda i: (0, 0)),
            pl.BlockSpec(w1z.shape, lambda i: (0, 0)),
            pl.BlockSpec(W2.shape, lambda i: (0, 0)),
            pl.BlockSpec(b2c.shape, lambda i: (0, 0)),
            pl.BlockSpec(W3.shape, lambda i: (0, 0)),
            pl.BlockSpec(b3c.shape, lambda i: (0, 0)),
            pl.BlockSpec((CB, 3), lambda i: (i, 0),
                         memory_space=pltpu.SMEM),
            pl.BlockSpec((CB, 1), lambda i: (i, 0)),
            pl.BlockSpec((CB, 1), lambda i: (i, 0)),
            pl.BlockSpec((CB, 1), lambda i: (i, 0)),
            pl.BlockSpec((1, N), lambda i: (0, 0)),
            pl.BlockSpec((1, N), lambda i: (0, 0)),
            pl.BlockSpec((1, N), lambda i: (0, 0)),
        ],
        out_specs=pl.BlockSpec((1, H3, CB), lambda i: (i, 0, 0)),
        out_shape=jax.ShapeDtypeStruct((C // CB, H3, CB), jnp.float32),
    )(XinT, W1, b1c, w1x, w1y, w1z, W2, b2c, W3, b3c,
      centers, cxc, cyc, czc, pxr, pyr, pzr)


def _tail_call(c2cols, c2smem, f2, w1c, w1f, b1, w2, b2, w3, b3,
               wa, wb, bb, g0, be0, w2f, b2f, g1, be1):
    C2, F2 = f2.shape
    OUT = w2f.shape[1]

    def body(cx_ref, cy_ref, cz_ref, cs_ref, f_ref, w1c_ref, w1f_ref,
             b1_ref, w2_ref, b2_ref, w3_ref, b3_ref, wa_ref, wb_ref,
             bb_ref, g0_ref, be0_ref, w2f_ref, b2f_ref, g1_ref, be1_ref,
             o_ref):
        dx = cx_ref[...] - cs_ref[0, 0]
        dy = cy_ref[...] - cs_ref[0, 1]
        dz = cz_ref[...] - cs_ref[0, 2]
        W1c = w1c_ref[...]
        dpart = dx * W1c[0:1] + dy * W1c[1:2] + dz * W1c[2:3]
        f2v = f_ref[...]
        h = jnp.maximum(
            dpart
            + jnp.dot(f2v, w1f_ref[...], preferred_element_type=jnp.float32)
            + b1_ref[...], 0.0)
        h = jnp.maximum(
            jnp.dot(h, w2_ref[...], preferred_element_type=jnp.float32)
            + b2_ref[...], 0.0)
        h = jnp.maximum(
            jnp.dot(h, w3_ref[...], preferred_element_type=jnp.float32)
            + b3_ref[...], 0.0)
        fm = jnp.max(h, axis=0, keepdims=True)
        kmw = jnp.dot(fm, wb_ref[...], preferred_element_type=jnp.float32)
        y = (jnp.dot(f2v, wa_ref[...], preferred_element_type=jnp.float32)
             + kmw + bb_ref[...])
        y = jnp.maximum(g0_ref[...] * y * _INV_BN + be0_ref[...], 0.0)
        y = jnp.dot(y, w2f_ref[...], preferred_element_type=jnp.float32) \
            + b2f_ref[...]
        y = jnp.maximum(g1_ref[...] * y * _INV_BN + be1_ref[...], 0.0)
        o_ref[...] = y

    vspec = lambda a: pl.BlockSpec(a.shape, lambda: (0,) * a.ndim)
    args = (*c2cols, c2smem, f2, w1c, w1f, b1, w2, b2, w3, b3,
            wa, wb, bb, g0, be0, w2f, b2f, g1, be1)
    in_specs = [vspec(a) for a in args]
    in_specs[3] = pl.BlockSpec(c2smem.shape, lambda: (0, 0),
                               memory_space=pltpu.SMEM)
    return pl.pallas_call(
        body,
        in_specs=in_specs,
        out_specs=pl.BlockSpec((C2, OUT), lambda: (0, 0)),
        out_shape=jax.ShapeDtypeStruct((C2, OUT), jnp.float32),
    )(*args)


_NC, _NS, _L = 2, 16, 16
_NW = _NC * _NS


def _knn4_sc(qcoords, pcoords):
    Q = qcoords.shape[0]
    P = pcoords.shape[0]
    L = _L
    nq = Q // _NW
    ng = nq // L
    f32 = jnp.float32
    i32 = jnp.int32

    qx = qcoords[:, 0]
    qy = qcoords[:, 1]
    qz = qcoords[:, 2]
    px_s = jnp.repeat(pcoords[:, 0], L)
    py_s = jnp.repeat(pcoords[:, 1], L)
    pz_s = jnp.repeat(pcoords[:, 2], L)

    out_type = ([jax.ShapeDtypeStruct((Q,), f32) for _ in range(4)]
                + [jax.ShapeDtypeStruct((Q,), i32) for _ in range(4)])
    scratch = ([pltpu.VMEM((nq,), f32) for _ in range(3)]
               + [pltpu.VMEM((P * L,), f32) for _ in range(3)]
               + [pltpu.VMEM((nq,), f32) for _ in range(4)]
               + [pltpu.VMEM((nq,), i32) for _ in range(4)])

    mesh = plsc.VectorSubcoreMesh(core_axis_name="c", subcore_axis_name="s")

    @functools.partial(pl.kernel, mesh=mesh, out_type=out_type,
                       scratch_types=scratch)
    def knn_kernel(qx_h, qy_h, qz_h, px_h, py_h, pz_h,
                   sq0_h, sq1_h, sq2_h, sq3_h,
                   ik0_h, ik1_h, ik2_h, ik3_h,
                   qxv, qyv, qzv, pxv, pyv, pzv,
                   t0v, t1v, t2v, t3v, i0v, i1v, i2v, i3v):
        wid = lax.axis_index("s") * _NC + lax.axis_index("c")
        base = wid * nq
        pltpu.sync_copy(qx_h.at[pl.ds(base, nq)], qxv)
        pltpu.sync_copy(qy_h.at[pl.ds(base, nq)], qyv)
        pltpu.sync_copy(qz_h.at[pl.ds(base, nq)], qzv)
        pltpu.sync_copy(px_h, pxv)
        pltpu.sync_copy(py_h, pyv)
        pltpu.sync_copy(pz_h, pzv)
        for g in range(ng):
            qxg = qxv[pl.ds(g * L, L)]
            qyg = qyv[pl.ds(g * L, L)]
            qzg = qzv[pl.ds(g * L, L)]
            inf16 = jnp.full((L,), np.float32(np.inf), f32)
            zero16 = jnp.zeros((L,), i32)
            state = (inf16, inf16, inf16, inf16,
                     zero16, zero16, zero16, zero16)

            def point_body(p, st, qxg=qxg, qyg=qyg, qzg=qzg):
                t0, t1, t2, t3, i0, i1, i2, i3 = st
                pxs = pxv[pl.ds(p * L, L)]
                pys = pyv[pl.ds(p * L, L)]
                pzs = pzv[pl.ds(p * L, L)]
                dx = qxg - pxs
                dy = qyg - pys
                dz = qzg - pzs
                dv = dx * dx + dy * dy + dz * dz
                iv = jnp.full((L,), 0, i32) + p
                c0 = dv < t0
                t0n = jnp.where(c0, dv, t0)
                i0n = jnp.where(c0, iv, i0)
                dv1 = jnp.where(c0, t0, dv)
                iv1 = jnp.where(c0, i0, iv)
                c1 = dv1 < t1
                t1n = jnp.where(c1, dv1, t1)
                i1n = jnp.where(c1, iv1, i1)
                dv2 = jnp.where(c1, t1, dv1)
                iv2 = jnp.where(c1, i1, iv1)
                c2 = dv2 < t2
                t2n = jnp.where(c2, dv2, t2)
                i2n = jnp.where(c2, iv2, i2)
                dv3 = jnp.where(c2, t2, dv2)
                iv3 = jnp.where(c2, i2, iv2)
                c3 = dv3 < t3
                t3n = jnp.where(c3, dv3, t3)
                i3n = jnp.where(c3, iv3, i3)
                return (t0n, t1n, t2n, t3n, i0n, i1n, i2n, i3n)

            st = lax.fori_loop(0, P, point_body, state, unroll=4)
            t0v[pl.ds(g * L, L)] = st[0]
            t1v[pl.ds(g * L, L)] = st[1]
            t2v[pl.ds(g * L, L)] = st[2]
            t3v[pl.ds(g * L, L)] = st[3]
            i0v[pl.ds(g * L, L)] = st[4]
            i1v[pl.ds(g * L, L)] = st[5]
            i2v[pl.ds(g * L, L)] = st[6]
            i3v[pl.ds(g * L, L)] = st[7]
        pltpu.sync_copy(t0v, sq0_h.at[pl.ds(base, nq)])
        pltpu.sync_copy(t1v, sq1_h.at[pl.ds(base, nq)])
        pltpu.sync_copy(t2v, sq2_h.at[pl.ds(base, nq)])
        pltpu.sync_copy(t3v, sq3_h.at[pl.ds(base, nq)])
        pltpu.sync_copy(i0v, ik0_h.at[pl.ds(base, nq)])
        pltpu.sync_copy(i1v, ik1_h.at[pl.ds(base, nq)])
        pltpu.sync_copy(i2v, ik2_h.at[pl.ds(base, nq)])
        pltpu.sync_copy(i3v, ik3_h.at[pl.ds(base, nq)])

    res = knn_kernel(qx, qy, qz, px_s, py_s, pz_s)
    return list(res[:4]), list(res[4:])


def _fp_call(sqs, iks_in, from_f, f_prev, layer_arrays, bn_flags, QB):
    Q = sqs[0].shape[0]
    P, F = from_f.shape
    Dprev = f_prev.shape[1]
    OUT = layer_arrays[-1][0].shape[1]

    flat = []
    for arrs in layer_arrays:
        flat.extend(arrs)
    n_flat = len(flat)

    def body(*refs):
        s_refs = refs[0:4]
        i_refs = refs[4:8]
        ff_ref = refs[8]
        fp_ref = refs[9]
        lrefs = list(refs[10:10 + n_flat])
        o_ref = refs[10 + n_flat]
        cand = []
        for k in range(4):
            sq = s_refs[k][...]
            z = sq == 0.0
            d = jnp.where(z, 0.0, jnp.sqrt(jnp.where(z, 1.0, sq)))
            cand.append((d, i_refs[k][...]))

        def cswap(a, b):
            da, ia = a
            db, ib = b
            sw = (da > db) | ((da == db) & (ia > ib))
            lo = (jnp.where(sw, db, da), jnp.where(sw, ib, ia))
            hi = (jnp.where(sw, da, db), jnp.where(sw, ia, ib))
            return lo, hi

        for a, b in [(0, 1), (2, 3), (0, 2), (1, 3), (1, 2)]:
            cand[a], cand[b] = cswap(cand[a], cand[b])
        dks = [cand[k][0] for k in range(3)]
        iks = [cand[k][1] for k in range(3)]
        iz = [dk == 0.0 for dk in dks]
        any_zero = iz[0] | iz[1] | iz[2]
        raws = []
        for z, dk in zip(iz, dks):
            safe = jnp.where(z, 1.0, dk)
            raws.append(1.0 / (safe * safe))
        s = raws[0] + raws[1] + raws[2]
        col = lax.broadcasted_iota(jnp.int32, (QB, P), 1)
        Wc = jnp.zeros((QB, P), jnp.float32)
        for k in range(3):
            wk = jnp.where(any_zero, iz[k].astype(jnp.float32), raws[k] / s)
            Wc = Wc + jnp.where(col == iks[k], wk, 0.0)
        km = jnp.dot(Wc, ff_ref[...], preferred_element_type=jnp.float32)
        wp = lrefs[0][...]
        wk_ = lrefs[1][...]
        b = lrefs[2][...]
        x = (jnp.dot(fp_ref[...], wp, preferred_element_type=jnp.float32)
             + jnp.dot(km, wk_, preferred_element_type=jnp.float32) + b)
        li = 3
        if bn_flags[0]:
            x = jnp.maximum(lrefs[li][...] * x * _INV_BN
                            + lrefs[li + 1][...], 0.0)
            li += 2
        for has_bn in bn_flags[1:]:
            w = lrefs[li][...]
            b = lrefs[li + 1][...]
            x = jnp.dot(x, w, preferred_element_type=jnp.float32) + b
            li += 2
            if has_bn:
                x = jnp.maximum(lrefs[li][...] * x * _INV_BN
                                + lrefs[li + 1][...], 0.0)
                li += 2
        o_ref[...] = x

    def full2(a):
        s = a.shape
        return pl.BlockSpec(s, lambda i: (0, 0))

    qspec = pl.BlockSpec((QB, 1), lambda i: (i, 0))
    in_specs = ([qspec] * 8 + [
        full2(from_f),
        pl.BlockSpec((QB, Dprev), lambda i: (i, 0)),
    ] + [full2(a) for a in flat])
    return pl.pallas_call(
        body,
        grid=(Q // QB,),
        in_specs=in_specs,
        out_specs=pl.BlockSpec((QB, OUT), lambda i: (i, 0)),
        out_shape=jax.ShapeDtypeStruct((Q, OUT), jnp.float32),
    )(*sqs, *iks_in, from_f, f_prev, *flat)


def kernel(coords, features, params):
    coords = coords.astype(jnp.float32)
    features = features.astype(jnp.float32)
    N = coords.shape[0]

    pxg = coords[:, 0].reshape(N // 128, 128)
    pyg = coords[:, 1].reshape(N // 128, 128)
    pzg = coords[:, 2].reshape(N // 128, 128)

    c1coords = _fps_call(pxg, pyg, pzg, coords.reshape(1, -1),
                         512).reshape(512, 3)
    c1x = c1coords[:, 0].reshape(4, 128)
    c1y = c1coords[:, 1].reshape(4, 128)
    c1z = c1coords[:, 2].reshape(4, 128)
    c2coords = _fps_call(c1x, c1y, c1z, c1coords.reshape(1, -1),
                         128).reshape(128, 3)
    c2x = c2coords[:, 0].reshape(1, 128)
    c2y = c2coords[:, 1].reshape(1, 128)
    c2z = c2coords[:, 2].reshape(1, 128)

    sq2s, ik2s = _knn4_sc(coords, c1coords)
    sq1s, ik1s = _knn4_sc(c1coords, c2coords)

    def _unblock(o):
        return jnp.transpose(o, (1, 0, 2)).reshape(o.shape[1], -1)

    sa0 = params['sa'][0]
    XinT1 = jnp.concatenate([coords.T, features.T], axis=0)
    f1T = _sa_call(
        XinT1, sa0['W1'], sa0['b1'][:, None],
        sa0['W1'][:, 0:1], sa0['W1'][:, 1:2], sa0['W1'][:, 2:3],
        sa0['W2'], sa0['b2'][:, None], sa0['W3'], sa0['b3'][:, None],
        c1coords,
        c1x.reshape(512, 1), c1y.reshape(512, 1), c1z.reshape(512, 1),
        coords[:, 0].reshape(1, N), coords[:, 1].reshape(1, N),
        coords[:, 2].reshape(1, N),
        0.2, CB=8)
    f1T = _unblock(f1T)
    f1 = f1T.T

    sa1 = params['sa'][1]
    c1coordsT = jnp.stack(
        [c1x.reshape(-1), c1y.reshape(-1), c1z.reshape(-1)], axis=0)
    XinT2 = jnp.concatenate([c1coordsT, f1T], axis=0)
    f2T = _sa_call(
        XinT2, sa1['W1'], sa1['b1'][:, None],
        sa1['W1'][:, 0:1], sa1['W1'][:, 1:2], sa1['W1'][:, 2:3],
        sa1['W2'], sa1['b2'][:, None], sa1['W3'], sa1['b3'][:, None],
        c2coords,
        c2x.reshape(128, 1), c2y.reshape(128, 1), c2z.reshape(128, 1),
        c1x.reshape(1, 512), c1y.reshape(1, 512), c1z.reshape(1, 512),
        0.4, CB=8)
    f2 = _unblock(f2T).T

    sa2 = params['sa'][2]
    fp0 = params['fp'][0]
    W1t = sa2['W1'].T
    fp0W0t = fp0[0]['W'].T
    g2 = _tail_call(
        (c2x.reshape(128, 1), c2y.reshape(128, 1), c2z.reshape(128, 1)),
        c2coords, f2,
        W1t[:3], W1t[3:], sa2['b1'][None, :],
        sa2['W2'].T, sa2['b2'][None, :], sa2['W3'].T, sa2['b3'][None, :],
        fp0W0t[:256], fp0W0t[256:], fp0[0]['b'][None, :],
        fp0[0]['gamma'][None, :], fp0[0]['beta'][None, :],
        fp0[1]['W'].T, fp0[1]['b'][None, :],
        fp0[1]['gamma'][None, :], fp0[1]['beta'][None, :])

    fp1 = params['fp'][1]
    W0t = fp1[0]['W'].T
    layer_arrays1 = [
        (W0t[:128], W0t[128:], fp1[0]['b'][None, :],
         fp1[0]['gamma'][None, :], fp1[0]['beta'][None, :]),
        (fp1[1]['W'].T, fp1[1]['b'][None, :],
         fp1[1]['gamma'][None, :], fp1[1]['beta'][None, :]),
    ]
    g1 = _fp_call(
        [a.reshape(512, 1) for a in sq1s],
        [a.reshape(512, 1) for a in ik1s],
        g2, f1, layer_arrays1, [True, True], QB=512)

    fp2 = params['fp'][2]
    W0t2 = fp2[0]['W'].T
    layer_arrays2 = [
        (W0t2[:3], W0t2[3:], fp2[0]['b'][None, :],
         fp2[0]['gamma'][None, :], fp2[0]['beta'][None, :]),
        (fp2[1]['W'].T, fp2[1]['b'][None, :],
         fp2[1]['gamma'][None, :], fp2[1]['beta'][None, :]),
        (fp2[2]['W'].T, fp2[2]['b'][None, :]),
    ]
    out = _fp_call(
        [a.reshape(4096, 1) for a in sq2s],
        [a.reshape(4096, 1) for a in ik2s],
        g1, features, layer_arrays2, [True, True, False], QB=512)
    return out


# speedup vs baseline: 1.1499x; 1.1222x over previous
import functools
import numpy as np
import jax
import jax.numpy as jnp
from jax import lax
from jax.experimental import pallas as pl
from jax.experimental.pallas import tpu as pltpu
from jax.experimental.pallas import tpu_sc as plsc

_INV_BN = np.float32(1.0) / np.sqrt(np.float32(1.0 + 1e-5))


def _flat_iota(shape):
    return (lax.broadcasted_iota(jnp.int32, shape, 0) * shape[1]
            + lax.broadcasted_iota(jnp.int32, shape, 1))


def _fps_call(pxg, pyg, pzg, pts_smem, K):
    R = pxg.shape[0]
    N = R * 128

    def body(px_ref, py_ref, pz_ref, ps_ref, o_ref):
        px = px_ref[...]
        py = py_ref[...]
        pz = pz_ref[...]
        fi = _flat_iota((R, 128))

        def dist_to(xj, yj, zj):
            dx = px - xj
            dy = py - yj
            dz = pz - zj
            return jnp.sqrt(dx * dx + dy * dy + dz * dz)

        x0 = ps_ref[0, 0]
        y0 = ps_ref[0, 1]
        z0 = ps_ref[0, 2]
        o_ref[0, 0] = x0
        o_ref[0, 1] = y0
        o_ref[0, 2] = z0

        def step(i, d):
            j = jnp.argmax(d).astype(jnp.int32)
            xj = ps_ref[0, j * 3]
            yj = ps_ref[0, j * 3 + 1]
            zj = ps_ref[0, j * 3 + 2]
            o_ref[0, i * 3] = xj
            o_ref[0, i * 3 + 1] = yj
            o_ref[0, i * 3 + 2] = zj
            return jnp.minimum(d, dist_to(xj, yj, zj))

        lax.fori_loop(1, K, step, dist_to(x0, y0, z0))

    return pl.pallas_call(
        body,
        in_specs=[
            pl.BlockSpec((R, 128), lambda: (0, 0)),
            pl.BlockSpec((R, 128), lambda: (0, 0)),
            pl.BlockSpec((R, 128), lambda: (0, 0)),
            pl.BlockSpec((1, 3 * N), lambda: (0, 0),
                         memory_space=pltpu.SMEM),
        ],
        out_specs=pl.BlockSpec((1, 3 * K), lambda: (0, 0),
                               memory_space=pltpu.SMEM),
        out_shape=jax.ShapeDtypeStruct((1, 3 * K), jnp.float32),
    )(pxg, pyg, pzg, pts_smem)


def _sa_call(XinT, W1, b1c, w1x, w1y, w1z, W2, b2c, W3, b3c,
             centers, cxc, cyc, czc, pxr, pyr, pzr, radius, CB):
    Din, N = XinT.shape
    C = cxc.shape[0]
    H3 = W3.shape[0]
    r32 = np.float32(radius)
    ninf = np.float32(-np.inf)

    def body(x_ref, w1_ref, b1_ref, w1x_ref, w1y_ref, w1z_ref,
             w2_ref, b2_ref, w3_ref, b3_ref, c_ref,
             cx_ref, cy_ref, cz_ref, px_ref, py_ref, pz_ref, o_ref):
        bf = jnp.bfloat16
        baseT = (jnp.dot(w1_ref[...], x_ref[...],
                         preferred_element_type=jnp.float32)
                 + b1_ref[...]).astype(bf)
        w1xv = w1x_ref[...]
        w1yv = w1y_ref[...]
        w1zv = w1z_ref[...]
        W2 = w2_ref[...].astype(bf)
        b2v = b2_ref[...].astype(bf)
        W3 = w3_ref[...].astype(bf)
        b3v = b3_ref[...]
        dx = cx_ref[...] - px_ref[...]
        dy = cy_ref[...] - py_ref[...]
        dz = cz_ref[...] - pz_ref[...]
        pen = jnp.where(
            jnp.sqrt(dx * dx + dy * dy + dz * dz) < r32, 0.0, ninf).astype(bf)
        li = lax.broadcasted_iota(jnp.int32, (H3, CB), 1)
        acc = jnp.zeros((H3, CB), jnp.float32)
        for c in range(CB):
            cxs = c_ref[c, 0]
            cys = c_ref[c, 1]
            czs = c_ref[c, 2]
            coffT = (cxs * w1xv + cys * w1yv + czs * w1zv).astype(bf)
            h = jnp.maximum(baseT - coffT, 0.0)
            h = jnp.maximum(
                jnp.dot(W2, h,
                        preferred_element_type=jnp.float32).astype(bf)
                + b2v, 0.0)
            h = jnp.dot(W3, h, preferred_element_type=jnp.float32).astype(bf)
            m = jnp.max(h + pen[c:c + 1, :], axis=1, keepdims=True)
            acc = jnp.where(li == c,
                            jnp.maximum(m.astype(jnp.float32) + b3v, 0.0),
                            acc)
        o_ref[0] = acc

    return pl.pallas_call(
        body,
        grid=(C // CB,),
        in_specs=[
            pl.BlockSpec((Din, N), lambda i: (0, 0)),
            pl.BlockSpec(W1.shape, lambda i: (0, 0)),
            pl.BlockSpec(b1c.shape, lambda i: (0, 0)),
            pl.BlockSpec(w1x.shape, lambda i: (0, 0)),
            pl.BlockSpec(w1y.shape, lambda i: (0, 0)),
            pl.BlockSpec(w1z.shape, lambda i: (0, 0)),
            pl.BlockSpec(W2.shape, lambda i: (0, 0)),
            pl.BlockSpec(b2c.shape, lambda i: (0, 0)),
            pl.BlockSpec(W3.shape, lambda i: (0, 0)),
            pl.BlockSpec(b3c.shape, lambda i: (0, 0)),
            pl.BlockSpec((CB, 3), lambda i: (i, 0),
                         memory_space=pltpu.SMEM),
            pl.BlockSpec((CB, 1), lambda i: (i, 0)),
            pl.BlockSpec((CB, 1), lambda i: (i, 0)),
            pl.BlockSpec((CB, 1), lambda i: (i, 0)),
            pl.BlockSpec((1, N), lambda i: (0, 0)),
            pl.BlockSpec((1, N), lambda i: (0, 0)),
            pl.BlockSpec((1, N), lambda i: (0, 0)),
        ],
        out_specs=pl.BlockSpec((1, H3, CB), lambda i: (i, 0, 0)),
        out_shape=jax.ShapeDtypeStruct((C // CB, H3, CB), jnp.float32),
    )(XinT, W1, b1c, w1x, w1y, w1z, W2, b2c, W3, b3c,
      centers, cxc, cyc, czc, pxr, pyr, pzr)


def _tail_call(c2cols, c2smem, f2, w1c, w1f, b1, w2, b2, w3, b3,
               wa, wb, bb, g0, be0, w2f, b2f, g1, be1):
    C2, F2 = f2.shape
    OUT = w2f.shape[1]

    def body(cx_ref, cy_ref, cz_ref, cs_ref, f_ref, w1c_ref, w1f_ref,
             b1_ref, w2_ref, b2_ref, w3_ref, b3_ref, wa_ref, wb_ref,
             bb_ref, g0_ref, be0_ref, w2f_ref, b2f_ref, g1_ref, be1_ref,
             o_ref):
        dx = cx_ref[...] - cs_ref[0, 0]
        dy = cy_ref[...] - cs_ref[0, 1]
        dz = cz_ref[...] - cs_ref[0, 2]
        W1c = w1c_ref[...]
        dpart = dx * W1c[0:1] + dy * W1c[1:2] + dz * W1c[2:3]
        f2v = f_ref[...]
        h = jnp.maximum(
            dpart
            + jnp.dot(f2v, w1f_ref[...], preferred_element_type=jnp.float32)
            + b1_ref[...], 0.0)
        h = jnp.maximum(
            jnp.dot(h, w2_ref[...], preferred_element_type=jnp.float32)
            + b2_ref[...], 0.0)
        h = jnp.maximum(
            jnp.dot(h, w3_ref[...], preferred_element_type=jnp.float32)
            + b3_ref[...], 0.0)
        fm = jnp.max(h, axis=0, keepdims=True)
        kmw = jnp.dot(fm, wb_ref[...], preferred_element_type=jnp.float32)
        y = (jnp.dot(f2v, wa_ref[...], preferred_element_type=jnp.float32)
             + kmw + bb_ref[...])
        y = jnp.maximum(g0_ref[...] * y * _INV_BN + be0_ref[...], 0.0)
        y = jnp.dot(y, w2f_ref[...], preferred_element_type=jnp.float32) \
            + b2f_ref[...]
        y = jnp.maximum(g1_ref[...] * y * _INV_BN + be1_ref[...], 0.0)
        o_ref[...] = y

    vspec = lambda a: pl.BlockSpec(a.shape, lambda: (0,) * a.ndim)
    args = (*c2cols, c2smem, f2, w1c, w1f, b1, w2, b2, w3, b3,
            wa, wb, bb, g0, be0, w2f, b2f, g1, be1)
    in_specs = [vspec(a) for a in args]
    in_specs[3] = pl.BlockSpec(c2smem.shape, lambda: (0, 0),
                               memory_space=pltpu.SMEM)
    return pl.pallas_call(
        body,
        in_specs=in_specs,
        out_specs=pl.BlockSpec((C2, OUT), lambda: (0, 0)),
        out_shape=jax.ShapeDtypeStruct((C2, OUT), jnp.float32),
    )(*args)


_NC, _NS, _L = 2, 16, 16
_NW = _NC * _NS


def _knn4_sc(qcoords, pcoords):
    Q = qcoords.shape[0]
    P = pcoords.shape[0]
    L = _L
    nq = Q // _NW
    ng = nq // L
    f32 = jnp.float32
    i32 = jnp.int32

    qx = qcoords[:, 0]
    qy = qcoords[:, 1]
    qz = qcoords[:, 2]
    px_s = jnp.repeat(pcoords[:, 0], L)
    py_s = jnp.repeat(pcoords[:, 1], L)
    pz_s = jnp.repeat(pcoords[:, 2], L)

    out_type = ([jax.ShapeDtypeStruct((Q,), f32) for _ in range(4)]
                + [jax.ShapeDtypeStruct((Q,), i32) for _ in range(4)])
    scratch = ([pltpu.VMEM((nq,), f32) for _ in range(3)]
               + [pltpu.VMEM((P * L,), f32) for _ in range(3)]
               + [pltpu.VMEM((nq,), f32) for _ in range(4)]
               + [pltpu.VMEM((nq,), i32) for _ in range(4)])

    mesh = plsc.VectorSubcoreMesh(core_axis_name="c", subcore_axis_name="s")

    @functools.partial(pl.kernel, mesh=mesh, out_type=out_type,
                       scratch_types=scratch)
    def knn_kernel(qx_h, qy_h, qz_h, px_h, py_h, pz_h,
                   sq0_h, sq1_h, sq2_h, sq3_h,
                   ik0_h, ik1_h, ik2_h, ik3_h,
                   qxv, qyv, qzv, pxv, pyv, pzv,
                   t0v, t1v, t2v, t3v, i0v, i1v, i2v, i3v):
        wid = lax.axis_index("s") * _NC + lax.axis_index("c")
        base = wid * nq
        pltpu.sync_copy(qx_h.at[pl.ds(base, nq)], qxv)
        pltpu.sync_copy(qy_h.at[pl.ds(base, nq)], qyv)
        pltpu.sync_copy(qz_h.at[pl.ds(base, nq)], qzv)
        pltpu.sync_copy(px_h, pxv)
        pltpu.sync_copy(py_h, pyv)
        pltpu.sync_copy(pz_h, pzv)
        for g in range(ng):
            qxg = qxv[pl.ds(g * L, L)]
            qyg = qyv[pl.ds(g * L, L)]
            qzg = qzv[pl.ds(g * L, L)]
            inf16 = jnp.full((L,), np.float32(np.inf), f32)
            zero16 = jnp.zeros((L,), i32)
            state = (inf16, inf16, inf16, inf16,
                     zero16, zero16, zero16, zero16)

            def point_body(p, st, qxg=qxg, qyg=qyg, qzg=qzg):
                t0, t1, t2, t3, i0, i1, i2, i3 = st
                pxs = pxv[pl.ds(p * L, L)]
                pys = pyv[pl.ds(p * L, L)]
                pzs = pzv[pl.ds(p * L, L)]
                dx = qxg - pxs
                dy = qyg - pys
                dz = qzg - pzs
                dv = dx * dx + dy * dy + dz * dz
                iv = jnp.full((L,), 0, i32) + p
                c0 = dv < t0
                t0n = jnp.where(c0, dv, t0)
                i0n = jnp.where(c0, iv, i0)
                dv1 = jnp.where(c0, t0, dv)
                iv1 = jnp.where(c0, i0, iv)
                c1 = dv1 < t1
                t1n = jnp.where(c1, dv1, t1)
                i1n = jnp.where(c1, iv1, i1)
                dv2 = jnp.where(c1, t1, dv1)
                iv2 = jnp.where(c1, i1, iv1)
                c2 = dv2 < t2
                t2n = jnp.where(c2, dv2, t2)
                i2n = jnp.where(c2, iv2, i2)
                dv3 = jnp.where(c2, t2, dv2)
                iv3 = jnp.where(c2, i2, iv2)
                c3 = dv3 < t3
                t3n = jnp.where(c3, dv3, t3)
                i3n = jnp.where(c3, iv3, i3)
                return (t0n, t1n, t2n, t3n, i0n, i1n, i2n, i3n)

            st = lax.fori_loop(0, P, point_body, state, unroll=4)
            t0v[pl.ds(g * L, L)] = st[0]
            t1v[pl.ds(g * L, L)] = st[1]
            t2v[pl.ds(g * L, L)] = st[2]
            t3v[pl.ds(g * L, L)] = st[3]
            i0v[pl.ds(g * L, L)] = st[4]
            i1v[pl.ds(g * L, L)] = st[5]
            i2v[pl.ds(g * L, L)] = st[6]
            i3v[pl.ds(g * L, L)] = st[7]
        pltpu.sync_copy(t0v, sq0_h.at[pl.ds(base, nq)])
        pltpu.sync_copy(t1v, sq1_h.at[pl.ds(base, nq)])
        pltpu.sync_copy(t2v, sq2_h.at[pl.ds(base, nq)])
        pltpu.sync_copy(t3v, sq3_h.at[pl.ds(base, nq)])
        pltpu.sync_copy(i0v, ik0_h.at[pl.ds(base, nq)])
        pltpu.sync_copy(i1v, ik1_h.at[pl.ds(base, nq)])
        pltpu.sync_copy(i2v, ik2_h.at[pl.ds(base, nq)])
        pltpu.sync_copy(i3v, ik3_h.at[pl.ds(base, nq)])

    res = knn_kernel(qx, qy, qz, px_s, py_s, pz_s)
    return list(res[:4]), list(res[4:])


def _fp_call(sqs, iks_in, from_f, f_prev, layer_arrays, bn_flags, QB):
    Q = sqs[0].shape[0]
    P, F = from_f.shape
    Dprev = f_prev.shape[1]
    OUT = layer_arrays[-1][0].shape[1]

    flat = []
    for arrs in layer_arrays:
        flat.extend(arrs)
    n_flat = len(flat)

    def body(*refs):
        s_refs = refs[0:4]
        i_refs = refs[4:8]
        ff_ref = refs[8]
        fp_ref = refs[9]
        lrefs = list(refs[10:10 + n_flat])
        o_ref = refs[10 + n_flat]
        cand = []
        for k in range(4):
            sq = s_refs[k][...]
            z = sq == 0.0
            d = jnp.where(z, 0.0, jnp.sqrt(jnp.where(z, 1.0, sq)))
            cand.append((d, i_refs[k][...]))

        def cswap(a, b):
            da, ia = a
            db, ib = b
            sw = (da > db) | ((da == db) & (ia > ib))
            lo = (jnp.where(sw, db, da), jnp.where(sw, ib, ia))
            hi = (jnp.where(sw, da, db), jnp.where(sw, ia, ib))
            return lo, hi

        for a, b in [(0, 1), (2, 3), (0, 2), (1, 3), (1, 2)]:
            cand[a], cand[b] = cswap(cand[a], cand[b])
        dks = [cand[k][0] for k in range(3)]
        iks = [cand[k][1] for k in range(3)]
        iz = [dk == 0.0 for dk in dks]
        any_zero = iz[0] | iz[1] | iz[2]
        raws = []
        for z, dk in zip(iz, dks):
            safe = jnp.where(z, 1.0, dk)
            raws.append(1.0 / (safe * safe))
        s = raws[0] + raws[1] + raws[2]
        col = lax.broadcasted_iota(jnp.int32, (QB, P), 1)
        Wc = jnp.zeros((QB, P), jnp.float32)
        for k in range(3):
            wk = jnp.where(any_zero, iz[k].astype(jnp.float32), raws[k] / s)
            Wc = Wc + jnp.where(col == iks[k], wk, 0.0)
        km = jnp.dot(Wc, ff_ref[...], preferred_element_type=jnp.float32)
        wp = lrefs[0][...]
        wk_ = lrefs[1][...]
        b = lrefs[2][...]
        x = (jnp.dot(fp_ref[...], wp, preferred_element_type=jnp.float32)
             + jnp.dot(km, wk_, preferred_element_type=jnp.float32) + b)
        li = 3
        if bn_flags[0]:
            x = jnp.maximum(lrefs[li][...] * x * _INV_BN
                            + lrefs[li + 1][...], 0.0)
            li += 2
        for has_bn in bn_flags[1:]:
            w = lrefs[li][...]
            b = lrefs[li + 1][...]
            x = jnp.dot(x, w, preferred_element_type=jnp.float32) + b
            li += 2
            if has_bn:
                x = jnp.maximum(lrefs[li][...] * x * _INV_BN
                                + lrefs[li + 1][...], 0.0)
                li += 2
        o_ref[...] = x

    def full2(a):
        s = a.shape
        return pl.BlockSpec(s, lambda i: (0, 0))

    qspec = pl.BlockSpec((QB, 1), lambda i: (i, 0))
    in_specs = ([qspec] * 8 + [
        full2(from_f),
        pl.BlockSpec((QB, Dprev), lambda i: (i, 0)),
    ] + [full2(a) for a in flat])
    return pl.pallas_call(
        body,
        grid=(Q // QB,),
        in_specs=in_specs,
        out_specs=pl.BlockSpec((QB, OUT), lambda i: (i, 0)),
        out_shape=jax.ShapeDtypeStruct((Q, OUT), jnp.float32),
    )(*sqs, *iks_in, from_f, f_prev, *flat)


def kernel(coords, features, params):
    coords = coords.astype(jnp.float32)
    features = features.astype(jnp.float32)
    N = coords.shape[0]

    pxg = coords[:, 0].reshape(N // 128, 128)
    pyg = coords[:, 1].reshape(N // 128, 128)
    pzg = coords[:, 2].reshape(N // 128, 128)

    c1coords = _fps_call(pxg, pyg, pzg, coords.reshape(1, -1),
                         512).reshape(512, 3)
    c1x = c1coords[:, 0].reshape(4, 128)
    c1y = c1coords[:, 1].reshape(4, 128)
    c1z = c1coords[:, 2].reshape(4, 128)
    c2coords = _fps_call(c1x, c1y, c1z, c1coords.reshape(1, -1),
                         128).reshape(128, 3)
    c2x = c2coords[:, 0].reshape(1, 128)
    c2y = c2coords[:, 1].reshape(1, 128)
    c2z = c2coords[:, 2].reshape(1, 128)

    sq2s, ik2s = _knn4_sc(coords, c1coords)
    sq1s, ik1s = _knn4_sc(c1coords, c2coords)

    def _unblock(o):
        return jnp.transpose(o, (1, 0, 2)).reshape(o.shape[1], -1)

    sa0 = params['sa'][0]
    XinT1 = jnp.concatenate([coords.T, features.T], axis=0)
    f1T = _sa_call(
        XinT1, sa0['W1'], sa0['b1'][:, None],
        sa0['W1'][:, 0:1], sa0['W1'][:, 1:2], sa0['W1'][:, 2:3],
        sa0['W2'], sa0['b2'][:, None], sa0['W3'], sa0['b3'][:, None],
        c1coords,
        c1x.reshape(512, 1), c1y.reshape(512, 1), c1z.reshape(512, 1),
        coords[:, 0].reshape(1, N), coords[:, 1].reshape(1, N),
        coords[:, 2].reshape(1, N),
        0.2, CB=8)
    f1T = _unblock(f1T)
    f1 = f1T.T

    sa1 = params['sa'][1]
    c1coordsT = jnp.stack(
        [c1x.reshape(-1), c1y.reshape(-1), c1z.reshape(-1)], axis=0)
    XinT2 = jnp.concatenate([c1coordsT, f1T], axis=0)
    f2T = _sa_call(
        XinT2, sa1['W1'], sa1['b1'][:, None],
        sa1['W1'][:, 0:1], sa1['W1'][:, 1:2], sa1['W1'][:, 2:3],
        sa1['W2'], sa1['b2'][:, None], sa1['W3'], sa1['b3'][:, None],
        c2coords,
        c2x.reshape(128, 1), c2y.reshape(128, 1), c2z.reshape(128, 1),
        c1x.reshape(1, 512), c1y.reshape(1, 512), c1z.reshape(1, 512),
        0.4, CB=8)
    f2 = _unblock(f2T).T

    sa2 = params['sa'][2]
    fp0 = params['fp'][0]
    W1t = sa2['W1'].T
    fp0W0t = fp0[0]['W'].T
    g2 = _tail_call(
        (c2x.reshape(128, 1), c2y.reshape(128, 1), c2z.reshape(128, 1)),
        c2coords, f2,
        W1t[:3], W1t[3:], sa2['b1'][None, :],
        sa2['W2'].T, sa2['b2'][None, :], sa2['W3'].T, sa2['b3'][None, :],
        fp0W0t[:256], fp0W0t[256:], fp0[0]['b'][None, :],
        fp0[0]['gamma'][None, :], fp0[0]['beta'][None, :],
        fp0[1]['W'].T, fp0[1]['b'][None, :],
        fp0[1]['gamma'][None, :], fp0[1]['beta'][None, :])

    fp1 = params['fp'][1]
    W0t = fp1[0]['W'].T
    layer_arrays1 = [
        (W0t[:128], W0t[128:], fp1[0]['b'][None, :],
         fp1[0]['gamma'][None, :], fp1[0]['beta'][None, :]),
        (fp1[1]['W'].T, fp1[1]['b'][None, :],
         fp1[1]['gamma'][None, :], fp1[1]['beta'][None, :]),
    ]
    g1 = _fp_call(
        [a.reshape(512, 1) for a in sq1s],
        [a.reshape(512, 1) for a in ik1s],
        g2, f1, layer_arrays1, [True, True], QB=512)

    fp2 = params['fp'][2]
    W0t2 = fp2[0]['W'].T
    layer_arrays2 = [
        (W0t2[:3], W0t2[3:], fp2[0]['b'][None, :],
         fp2[0]['gamma'][None, :], fp2[0]['beta'][None, :]),
        (fp2[1]['W'].T, fp2[1]['b'][None, :],
         fp2[1]['gamma'][None, :], fp2[1]['beta'][None, :]),
        (fp2[2]['W'].T, fp2[2]['b'][None, :]),
    ]
    out = _fp_call(
        [a.reshape(4096, 1) for a in sq2s],
        [a.reshape(4096, 1) for a in ik2s],
        g1, features, layer_arrays2, [True, True, False], QB=512)
    return out


# speedup vs baseline: 1.1611x; 1.0097x over previous
import functools
import numpy as np
import jax
import jax.numpy as jnp
from jax import lax
from jax.experimental import pallas as pl
from jax.experimental.pallas import tpu as pltpu
from jax.experimental.pallas import tpu_sc as plsc

_INV_BN = np.float32(1.0) / np.sqrt(np.float32(1.0 + 1e-5))


def _flat_iota(shape):
    return (lax.broadcasted_iota(jnp.int32, shape, 0) * shape[1]
            + lax.broadcasted_iota(jnp.int32, shape, 1))


def _fps_pair_call(pxg, pyg, pzg, pts_smem, K1, K2):
    R = pxg.shape[0]
    R2 = K1 // 128

    def one_level(px, py, pz, shape, ps_read, o_ref, K, accum):
        fi = _flat_iota(shape)

        def dist_to(xj, yj, zj):
            dx = px - xj
            dy = py - yj
            dz = pz - zj
            return jnp.sqrt(dx * dx + dy * dy + dz * dz)

        x0, y0, z0 = ps_read(jnp.int32(0))
        o_ref[0, 0] = x0
        o_ref[0, 1] = y0
        o_ref[0, 2] = z0
        if accum is not None:
            accum(jnp.int32(0), x0, y0, z0)

        def step(i, d):
            j = jnp.argmax(d).astype(jnp.int32)
            xj, yj, zj = ps_read(j)
            o_ref[0, i * 3] = xj
            o_ref[0, i * 3 + 1] = yj
            o_ref[0, i * 3 + 2] = zj
            if accum is not None:
                accum(i, xj, yj, zj)
            return jnp.minimum(d, dist_to(xj, yj, zj))

        lax.fori_loop(1, K, step, dist_to(x0, y0, z0))

    def body(px_ref, py_ref, pz_ref, ps_ref, o1_ref, o2_ref,
             cx_s, cy_s, cz_s):
        ki = _flat_iota((R2, 128))

        def read1(j):
            return (ps_ref[0, j * 3], ps_ref[0, j * 3 + 1],
                    ps_ref[0, j * 3 + 2])

        def accum1(i, xj, yj, zj):
            sel = ki == i
            cx_s[...] = jnp.where(sel, xj, cx_s[...])
            cy_s[...] = jnp.where(sel, yj, cy_s[...])
            cz_s[...] = jnp.where(sel, zj, cz_s[...])

        one_level(px_ref[...], py_ref[...], pz_ref[...], (R, 128),
                  read1, o1_ref, K1, accum1)

        def read2(j):
            return (o1_ref[0, j * 3], o1_ref[0, j * 3 + 1],
                    o1_ref[0, j * 3 + 2])

        one_level(cx_s[...], cy_s[...], cz_s[...], (R2, 128),
                  read2, o2_ref, K2, None)

    return pl.pallas_call(
        body,
        in_specs=[
            pl.BlockSpec((R, 128), lambda: (0, 0)),
            pl.BlockSpec((R, 128), lambda: (0, 0)),
            pl.BlockSpec((R, 128), lambda: (0, 0)),
            pl.BlockSpec((1, 3 * R * 128), lambda: (0, 0),
                         memory_space=pltpu.SMEM),
        ],
        out_specs=[
            pl.BlockSpec((1, 3 * K1), lambda: (0, 0),
                         memory_space=pltpu.SMEM),
            pl.BlockSpec((1, 3 * K2), lambda: (0, 0),
                         memory_space=pltpu.SMEM),
        ],
        out_shape=[jax.ShapeDtypeStruct((1, 3 * K1), jnp.float32),
                   jax.ShapeDtypeStruct((1, 3 * K2), jnp.float32)],
        scratch_shapes=[pltpu.VMEM((R2, 128), jnp.float32)] * 3,
    )(pxg, pyg, pzg, pts_smem)


def _sa_call(XinT, W1, b1c, w1x, w1y, w1z, W2, b2c, W3, b3c,
             centers, cxc, cyc, czc, pxr, pyr, pzr, radius, CB):
    Din, N = XinT.shape
    C = cxc.shape[0]
    H3 = W3.shape[0]
    r32 = np.float32(radius)
    ninf = np.float32(-np.inf)

    def body(x_ref, w1_ref, b1_ref, w1x_ref, w1y_ref, w1z_ref,
             w2_ref, b2_ref, w3_ref, b3_ref, c_ref,
             cx_ref, cy_ref, cz_ref, px_ref, py_ref, pz_ref, o_ref):
        bf = jnp.bfloat16
        baseT = (jnp.dot(w1_ref[...], x_ref[...],
                         preferred_element_type=jnp.float32)
                 + b1_ref[...]).astype(bf)
        w1xv = w1x_ref[...]
        w1yv = w1y_ref[...]
        w1zv = w1z_ref[...]
        W2 = w2_ref[...].astype(bf)
        b2v = b2_ref[...].astype(bf)
        W3 = w3_ref[...].astype(bf)
        b3v = b3_ref[...]
        dx = cx_ref[...] - px_ref[...]
        dy = cy_ref[...] - py_ref[...]
        dz = cz_ref[...] - pz_ref[...]
        pen = jnp.where(
            jnp.sqrt(dx * dx + dy * dy + dz * dz) < r32, 0.0, ninf).astype(bf)
        li = lax.broadcasted_iota(jnp.int32, (H3, CB), 1)
        acc = jnp.zeros((H3, CB), jnp.float32)
        for c in range(CB):
            cxs = c_ref[c, 0]
            cys = c_ref[c, 1]
            czs = c_ref[c, 2]
            coffT = (cxs * w1xv + cys * w1yv + czs * w1zv).astype(bf)
            h = jnp.maximum(baseT - coffT, 0.0)
            h = jnp.maximum(
                jnp.dot(W2, h,
                        preferred_element_type=jnp.float32).astype(bf)
                + b2v, 0.0)
            h = jnp.dot(W3, h, preferred_element_type=jnp.float32).astype(bf)
            m = jnp.max(h + pen[c:c + 1, :], axis=1, keepdims=True)
            acc = jnp.where(li == c,
                            jnp.maximum(m.astype(jnp.float32) + b3v, 0.0),
                            acc)
        o_ref[0] = acc

    return pl.pallas_call(
        body,
        grid=(C // CB,),
        in_specs=[
            pl.BlockSpec((Din, N), lambda i: (0, 0)),
            pl.BlockSpec(W1.shape, lambda i: (0, 0)),
            pl.BlockSpec(b1c.shape, lambda i: (0, 0)),
            pl.BlockSpec(w1x.shape, lambda i: (0, 0)),
            pl.BlockSpec(w1y.shape, lambda i: (0, 0)),
            pl.BlockSpec(w1z.shape, lambda i: (0, 0)),
            pl.BlockSpec(W2.shape, lambda i: (0, 0)),
            pl.BlockSpec(b2c.shape, lambda i: (0, 0)),
            pl.BlockSpec(W3.shape, lambda i: (0, 0)),
            pl.BlockSpec(b3c.shape, lambda i: (0, 0)),
            pl.BlockSpec((CB, 3), lambda i: (i, 0),
                         memory_space=pltpu.SMEM),
            pl.BlockSpec((CB, 1), lambda i: (i, 0)),
            pl.BlockSpec((CB, 1), lambda i: (i, 0)),
            pl.BlockSpec((CB, 1), lambda i: (i, 0)),
            pl.BlockSpec((1, N), lambda i: (0, 0)),
            pl.BlockSpec((1, N), lambda i: (0, 0)),
            pl.BlockSpec((1, N), lambda i: (0, 0)),
        ],
        out_specs=pl.BlockSpec((1, H3, CB), lambda i: (i, 0, 0)),
        out_shape=jax.ShapeDtypeStruct((C // CB, H3, CB), jnp.float32),
    )(XinT, W1, b1c, w1x, w1y, w1z, W2, b2c, W3, b3c,
      centers, cxc, cyc, czc, pxr, pyr, pzr)


def _tail_call(c2cols, c2smem, f2, w1c, w1f, b1, w2, b2, w3, b3,
               wa, wb, bb, g0, be0, w2f, b2f, g1, be1):
    C2, F2 = f2.shape
    OUT = w2f.shape[1]

    def body(cx_ref, cy_ref, cz_ref, cs_ref, f_ref, w1c_ref, w1f_ref,
             b1_ref, w2_ref, b2_ref, w3_ref, b3_ref, wa_ref, wb_ref,
             bb_ref, g0_ref, be0_ref, w2f_ref, b2f_ref, g1_ref, be1_ref,
             o_ref):
        dx = cx_ref[...] - cs_ref[0, 0]
        dy = cy_ref[...] - cs_ref[0, 1]
        dz = cz_ref[...] - cs_ref[0, 2]
        W1c = w1c_ref[...]
        dpart = dx * W1c[0:1] + dy * W1c[1:2] + dz * W1c[2:3]
        f2v = f_ref[...]
        h = jnp.maximum(
            dpart
            + jnp.dot(f2v, w1f_ref[...], preferred_element_type=jnp.float32)
            + b1_ref[...], 0.0)
        h = jnp.maximum(
            jnp.dot(h, w2_ref[...], preferred_element_type=jnp.float32)
            + b2_ref[...], 0.0)
        h = jnp.maximum(
            jnp.dot(h, w3_ref[...], preferred_element_type=jnp.float32)
            + b3_ref[...], 0.0)
        fm = jnp.max(h, axis=0, keepdims=True)
        kmw = jnp.dot(fm, wb_ref[...], preferred_element_type=jnp.float32)
        y = (jnp.dot(f2v, wa_ref[...], preferred_element_type=jnp.float32)
             + kmw + bb_ref[...])
        y = jnp.maximum(g0_ref[...] * y * _INV_BN + be0_ref[...], 0.0)
        y = jnp.dot(y, w2f_ref[...], preferred_element_type=jnp.float32) \
            + b2f_ref[...]
        y = jnp.maximum(g1_ref[...] * y * _INV_BN + be1_ref[...], 0.0)
        o_ref[...] = y

    vspec = lambda a: pl.BlockSpec(a.shape, lambda: (0,) * a.ndim)
    args = (*c2cols, c2smem, f2, w1c, w1f, b1, w2, b2, w3, b3,
            wa, wb, bb, g0, be0, w2f, b2f, g1, be1)
    in_specs = [vspec(a) for a in args]
    in_specs[3] = pl.BlockSpec(c2smem.shape, lambda: (0, 0),
                               memory_space=pltpu.SMEM)
    return pl.pallas_call(
        body,
        in_specs=in_specs,
        out_specs=pl.BlockSpec((C2, OUT), lambda: (0, 0)),
        out_shape=jax.ShapeDtypeStruct((C2, OUT), jnp.float32),
    )(*args)


_NC, _NS, _L = 2, 16, 16
_NW = _NC * _NS


def _knn4_sc_pair(stage_a, stage_b):
    f32 = jnp.float32
    i32 = jnp.int32
    L = _L
    stages = []
    args = []
    out_type = []
    scratch = []
    for qcoords, pcoords in (stage_a, stage_b):
        Q = qcoords.shape[0]
        P = pcoords.shape[0]
        nq = Q // _NW
        stages.append((Q, P, nq, nq // L))
        args += [qcoords[:, 0], qcoords[:, 1], qcoords[:, 2],
                 jnp.repeat(pcoords[:, 0], L),
                 jnp.repeat(pcoords[:, 1], L),
                 jnp.repeat(pcoords[:, 2], L)]
        out_type += ([jax.ShapeDtypeStruct((Q,), f32) for _ in range(4)]
                     + [jax.ShapeDtypeStruct((Q,), i32) for _ in range(4)])
        scratch += ([pltpu.VMEM((nq,), f32) for _ in range(3)]
                    + [pltpu.VMEM((P * L,), f32) for _ in range(3)]
                    + [pltpu.VMEM((nq,), f32) for _ in range(4)]
                    + [pltpu.VMEM((nq,), i32) for _ in range(4)])

    mesh = plsc.VectorSubcoreMesh(core_axis_name="c", subcore_axis_name="s")

    @functools.partial(pl.kernel, mesh=mesh, out_type=out_type,
                       scratch_types=scratch)
    def knn_kernel(*refs):
        in_refs = refs[:12]
        out_refs = refs[12:28]
        scr_refs = refs[28:]
        wid = lax.axis_index("s") * _NC + lax.axis_index("c")
        for si, (Q, P, nq, ng) in enumerate(stages):
            qx_h, qy_h, qz_h, px_h, py_h, pz_h = in_refs[si * 6:si * 6 + 6]
            (sq0_h, sq1_h, sq2_h, sq3_h,
             ik0_h, ik1_h, ik2_h, ik3_h) = out_refs[si * 8:si * 8 + 8]
            (qxv, qyv, qzv, pxv, pyv, pzv,
             t0v, t1v, t2v, t3v, i0v, i1v, i2v, i3v) = \
                scr_refs[si * 14:si * 14 + 14]
            base = wid * nq
            pltpu.sync_copy(qx_h.at[pl.ds(base, nq)], qxv)
            pltpu.sync_copy(qy_h.at[pl.ds(base, nq)], qyv)
            pltpu.sync_copy(qz_h.at[pl.ds(base, nq)], qzv)
            pltpu.sync_copy(px_h, pxv)
            pltpu.sync_copy(py_h, pyv)
            pltpu.sync_copy(pz_h, pzv)
            for g in range(ng):
                qxg = qxv[pl.ds(g * L, L)]
                qyg = qyv[pl.ds(g * L, L)]
                qzg = qzv[pl.ds(g * L, L)]
                inf16 = jnp.full((L,), np.float32(np.inf), f32)
                zero16 = jnp.zeros((L,), i32)
                state = (inf16, inf16, inf16, inf16,
                         zero16, zero16, zero16, zero16)

                def point_body(p, st, qxg=qxg, qyg=qyg, qzg=qzg,
                               pxv=pxv, pyv=pyv, pzv=pzv):
                    t0, t1, t2, t3, i0, i1, i2, i3 = st
                    pxs = pxv[pl.ds(p * L, L)]
                    pys = pyv[pl.ds(p * L, L)]
                    pzs = pzv[pl.ds(p * L, L)]
                    dx = qxg - pxs
                    dy = qyg - pys
                    dz = qzg - pzs
                    dv = dx * dx + dy * dy + dz * dz
                    iv = jnp.full((L,), 0, i32) + p
                    c0 = dv < t0
                    t0n = jnp.where(c0, dv, t0)
                    i0n = jnp.where(c0, iv, i0)
                    dv1 = jnp.where(c0, t0, dv)
                    iv1 = jnp.where(c0, i0, iv)
                    c1 = dv1 < t1
                    t1n = jnp.where(c1, dv1, t1)
                    i1n = jnp.where(c1, iv1, i1)
                    dv2 = jnp.where(c1, t1, dv1)
                    iv2 = jnp.where(c1, i1, iv1)
                    c2 = dv2 < t2
                    t2n = jnp.where(c2, dv2, t2)
                    i2n = jnp.where(c2, iv2, i2)
                    dv3 = jnp.where(c2, t2, dv2)
                    iv3 = jnp.where(c2, i2, iv2)
                    c3 = dv3 < t3
                    t3n = jnp.where(c3, dv3, t3)
                    i3n = jnp.where(c3, iv3, i3)
                    return (t0n, t1n, t2n, t3n, i0n, i1n, i2n, i3n)

                st = lax.fori_loop(0, P, point_body, state, unroll=4)
                t0v[pl.ds(g * L, L)] = st[0]
                t1v[pl.ds(g * L, L)] = st[1]
                t2v[pl.ds(g * L, L)] = st[2]
                t3v[pl.ds(g * L, L)] = st[3]
                i0v[pl.ds(g * L, L)] = st[4]
                i1v[pl.ds(g * L, L)] = st[5]
                i2v[pl.ds(g * L, L)] = st[6]
                i3v[pl.ds(g * L, L)] = st[7]
            pltpu.sync_copy(t0v, sq0_h.at[pl.ds(base, nq)])
            pltpu.sync_copy(t1v, sq1_h.at[pl.ds(base, nq)])
            pltpu.sync_copy(t2v, sq2_h.at[pl.ds(base, nq)])
            pltpu.sync_copy(t3v, sq3_h.at[pl.ds(base, nq)])
            pltpu.sync_copy(i0v, ik0_h.at[pl.ds(base, nq)])
            pltpu.sync_copy(i1v, ik1_h.at[pl.ds(base, nq)])
            pltpu.sync_copy(i2v, ik2_h.at[pl.ds(base, nq)])
            pltpu.sync_copy(i3v, ik3_h.at[pl.ds(base, nq)])

    res = knn_kernel(*args)
    return ((list(res[0:4]), list(res[4:8])),
            (list(res[8:12]), list(res[12:16])))


def _fp_call(sqs, iks_in, from_f, f_prev, layer_arrays, bn_flags, QB):
    Q = sqs[0].shape[0]
    P, F = from_f.shape
    Dprev = f_prev.shape[1]
    OUT = layer_arrays[-1][0].shape[1]

    flat = []
    for arrs in layer_arrays:
        flat.extend(arrs)
    n_flat = len(flat)

    def body(*refs):
        s_refs = refs[0:4]
        i_refs = refs[4:8]
        ff_ref = refs[8]
        fp_ref = refs[9]
        lrefs = list(refs[10:10 + n_flat])
        o_ref = refs[10 + n_flat]
        cand = []
        for k in range(4):
            sq = s_refs[k][...]
            z = sq == 0.0
            d = jnp.where(z, 0.0, jnp.sqrt(jnp.where(z, 1.0, sq)))
            cand.append((d, i_refs[k][...]))

        def cswap(a, b):
            da, ia = a
            db, ib = b
            sw = (da > db) | ((da == db) & (ia > ib))
            lo = (jnp.where(sw, db, da), jnp.where(sw, ib, ia))
            hi = (jnp.where(sw, da, db), jnp.where(sw, ia, ib))
            return lo, hi

        for a, b in [(0, 1), (2, 3), (0, 2), (1, 3), (1, 2)]:
            cand[a], cand[b] = cswap(cand[a], cand[b])
        dks = [cand[k][0] for k in range(3)]
        iks = [cand[k][1] for k in range(3)]
        iz = [dk == 0.0 for dk in dks]
        any_zero = iz[0] | iz[1] | iz[2]
        raws = []
        for z, dk in zip(iz, dks):
            safe = jnp.where(z, 1.0, dk)
            raws.append(1.0 / (safe * safe))
        s = raws[0] + raws[1] + raws[2]
        col = lax.broadcasted_iota(jnp.int32, (QB, P), 1)
        Wc = jnp.zeros((QB, P), jnp.float32)
        for k in range(3):
            wk = jnp.where(any_zero, iz[k].astype(jnp.float32), raws[k] / s)
            Wc = Wc + jnp.where(col == iks[k], wk, 0.0)
        km = jnp.dot(Wc, ff_ref[...], preferred_element_type=jnp.float32)
        wp = lrefs[0][...]
        wk_ = lrefs[1][...]
        b = lrefs[2][...]
        x = (jnp.dot(fp_ref[...], wp, preferred_element_type=jnp.float32)
             + jnp.dot(km, wk_, preferred_element_type=jnp.float32) + b)
        li = 3
        if bn_flags[0]:
            x = jnp.maximum(lrefs[li][...] * x * _INV_BN
                            + lrefs[li + 1][...], 0.0)
            li += 2
        for has_bn in bn_flags[1:]:
            w = lrefs[li][...]
            b = lrefs[li + 1][...]
            x = jnp.dot(x, w, preferred_element_type=jnp.float32) + b
            li += 2
            if has_bn:
                x = jnp.maximum(lrefs[li][...] * x * _INV_BN
                                + lrefs[li + 1][...], 0.0)
                li += 2
        o_ref[...] = x

    def full2(a):
        s = a.shape
        return pl.BlockSpec(s, lambda i: (0, 0))

    qspec = pl.BlockSpec((QB, 1), lambda i: (i, 0))
    in_specs = ([qspec] * 8 + [
        full2(from_f),
        pl.BlockSpec((QB, Dprev), lambda i: (i, 0)),
    ] + [full2(a) for a in flat])
    return pl.pallas_call(
        body,
        grid=(Q // QB,),
        in_specs=in_specs,
        out_specs=pl.BlockSpec((QB, OUT), lambda i: (i, 0)),
        out_shape=jax.ShapeDtypeStruct((Q, OUT), jnp.float32),
    )(*sqs, *iks_in, from_f, f_prev, *flat)


def kernel(coords, features, params):
    coords = coords.astype(jnp.float32)
    features = features.astype(jnp.float32)
    N = coords.shape[0]

    pxg = coords[:, 0].reshape(N // 128, 128)
    pyg = coords[:, 1].reshape(N // 128, 128)
    pzg = coords[:, 2].reshape(N // 128, 128)

    c1flat, c2flat = _fps_pair_call(pxg, pyg, pzg, coords.reshape(1, -1),
                                    512, 128)
    c1coords = c1flat.reshape(512, 3)
    c2coords = c2flat.reshape(128, 3)
    c1x = c1coords[:, 0].reshape(4, 128)
    c1y = c1coords[:, 1].reshape(4, 128)
    c1z = c1coords[:, 2].reshape(4, 128)
    c2x = c2coords[:, 0].reshape(1, 128)
    c2y = c2coords[:, 1].reshape(1, 128)
    c2z = c2coords[:, 2].reshape(1, 128)

    (sq2s, ik2s), (sq1s, ik1s) = _knn4_sc_pair(
        (coords, c1coords), (c1coords, c2coords))

    def _unblock(o):
        return jnp.transpose(o, (1, 0, 2)).reshape(o.shape[1], -1)

    sa0 = params['sa'][0]
    XinT1 = jnp.concatenate([coords.T, features.T], axis=0)
    f1T = _sa_call(
        XinT1, sa0['W1'], sa0['b1'][:, None],
        sa0['W1'][:, 0:1], sa0['W1'][:, 1:2], sa0['W1'][:, 2:3],
        sa0['W2'], sa0['b2'][:, None], sa0['W3'], sa0['b3'][:, None],
        c1coords,
        c1x.reshape(512, 1), c1y.reshape(512, 1), c1z.reshape(512, 1),
        coords[:, 0].reshape(1, N), coords[:, 1].reshape(1, N),
        coords[:, 2].reshape(1, N),
        0.2, CB=8)
    f1T = _unblock(f1T)
    f1 = f1T.T

    sa1 = params['sa'][1]
    c1coordsT = jnp.stack(
        [c1x.reshape(-1), c1y.reshape(-1), c1z.reshape(-1)], axis=0)
    XinT2 = jnp.concatenate([c1coordsT, f1T], axis=0)
    f2T = _sa_call(
        XinT2, sa1['W1'], sa1['b1'][:, None],
        sa1['W1'][:, 0:1], sa1['W1'][:, 1:2], sa1['W1'][:, 2:3],
        sa1['W2'], sa1['b2'][:, None], sa1['W3'], sa1['b3'][:, None],
        c2coords,
        c2x.reshape(128, 1), c2y.reshape(128, 1), c2z.reshape(128, 1),
        c1x.reshape(1, 512), c1y.reshape(1, 512), c1z.reshape(1, 512),
        0.4, CB=8)
    f2 = _unblock(f2T).T

    sa2 = params['sa'][2]
    fp0 = params['fp'][0]
    W1t = sa2['W1'].T
    fp0W0t = fp0[0]['W'].T
    g2 = _tail_call(
        (c2x.reshape(128, 1), c2y.reshape(128, 1), c2z.reshape(128, 1)),
        c2coords, f2,
        W1t[:3], W1t[3:], sa2['b1'][None, :],
        sa2['W2'].T, sa2['b2'][None, :], sa2['W3'].T, sa2['b3'][None, :],
        fp0W0t[:256], fp0W0t[256:], fp0[0]['b'][None, :],
        fp0[0]['gamma'][None, :], fp0[0]['beta'][None, :],
        fp0[1]['W'].T, fp0[1]['b'][None, :],
        fp0[1]['gamma'][None, :], fp0[1]['beta'][None, :])

    fp1 = params['fp'][1]
    W0t = fp1[0]['W'].T
    layer_arrays1 = [
        (W0t[:128], W0t[128:], fp1[0]['b'][None, :],
         fp1[0]['gamma'][None, :], fp1[0]['beta'][None, :]),
        (fp1[1]['W'].T, fp1[1]['b'][None, :],
         fp1[1]['gamma'][None, :], fp1[1]['beta'][None, :]),
    ]
    g1 = _fp_call(
        [a.reshape(512, 1) for a in sq1s],
        [a.reshape(512, 1) for a in ik1s],
        g2, f1, layer_arrays1, [True, True], QB=512)

    fp2 = params['fp'][2]
    W0t2 = fp2[0]['W'].T
    layer_arrays2 = [
        (W0t2[:3], W0t2[3:], fp2[0]['b'][None, :],
         fp2[0]['gamma'][None, :], fp2[0]['beta'][None, :]),
        (fp2[1]['W'].T, fp2[1]['b'][None, :],
         fp2[1]['gamma'][None, :], fp2[1]['beta'][None, :]),
        (fp2[2]['W'].T, fp2[2]['b'][None, :]),
    ]
    out = _fp_call(
        [a.reshape(4096, 1) for a in sq2s],
        [a.reshape(4096, 1) for a in ik2s],
        g1, features, layer_arrays2, [True, True, False], QB=512)
    return out


# speedup vs baseline: 1.2062x; 1.0388x over previous
import functools
import numpy as np
import jax
import jax.numpy as jnp
from jax import lax
from jax.experimental import pallas as pl
from jax.experimental.pallas import tpu as pltpu
from jax.experimental.pallas import tpu_sc as plsc

_INV_BN = np.float32(1.0) / np.sqrt(np.float32(1.0 + 1e-5))


def _flat_iota(shape):
    return (lax.broadcasted_iota(jnp.int32, shape, 0) * shape[1]
            + lax.broadcasted_iota(jnp.int32, shape, 1))


def _fps_pair_call(pxg, pyg, pzg, pts_smem, K1, K2):
    R = pxg.shape[0]
    R2 = K1 // 128

    def one_level(px, py, pz, shape, ps_read, o_ref, K, accum):
        fi = _flat_iota(shape)

        def dist_to(xj, yj, zj):
            dx = px - xj
            dy = py - yj
            dz = pz - zj
            return jnp.sqrt(dx * dx + dy * dy + dz * dz)

        x0, y0, z0 = ps_read(jnp.int32(0))
        o_ref[0, 0] = x0
        o_ref[0, 1] = y0
        o_ref[0, 2] = z0
        if accum is not None:
            accum(jnp.int32(0), x0, y0, z0)

        def step(i, d):
            j = jnp.argmax(d).astype(jnp.int32)
            xj, yj, zj = ps_read(j)
            o_ref[0, i * 3] = xj
            o_ref[0, i * 3 + 1] = yj
            o_ref[0, i * 3 + 2] = zj
            if accum is not None:
                accum(i, xj, yj, zj)
            return jnp.minimum(d, dist_to(xj, yj, zj))

        lax.fori_loop(1, K, step, dist_to(x0, y0, z0))

    def body(px_ref, py_ref, pz_ref, ps_ref, o1_ref, o2_ref,
             cx_s, cy_s, cz_s):
        ki = _flat_iota((R2, 128))

        def read1(j):
            return (ps_ref[0, j * 3], ps_ref[0, j * 3 + 1],
                    ps_ref[0, j * 3 + 2])

        def accum1(i, xj, yj, zj):
            sel = ki == i
            cx_s[...] = jnp.where(sel, xj, cx_s[...])
            cy_s[...] = jnp.where(sel, yj, cy_s[...])
            cz_s[...] = jnp.where(sel, zj, cz_s[...])

        one_level(px_ref[...], py_ref[...], pz_ref[...], (R, 128),
                  read1, o1_ref, K1, accum1)

        def read2(j):
            return (o1_ref[0, j * 3], o1_ref[0, j * 3 + 1],
                    o1_ref[0, j * 3 + 2])

        one_level(cx_s[...], cy_s[...], cz_s[...], (R2, 128),
                  read2, o2_ref, K2, None)

    return pl.pallas_call(
        body,
        in_specs=[
            pl.BlockSpec((R, 128), lambda: (0, 0)),
            pl.BlockSpec((R, 128), lambda: (0, 0)),
            pl.BlockSpec((R, 128), lambda: (0, 0)),
            pl.BlockSpec((1, 3 * R * 128), lambda: (0, 0),
                         memory_space=pltpu.SMEM),
        ],
        out_specs=[
            pl.BlockSpec((1, 3 * K1), lambda: (0, 0),
                         memory_space=pltpu.SMEM),
            pl.BlockSpec((1, 3 * K2), lambda: (0, 0),
                         memory_space=pltpu.SMEM),
        ],
        out_shape=[jax.ShapeDtypeStruct((1, 3 * K1), jnp.float32),
                   jax.ShapeDtypeStruct((1, 3 * K2), jnp.float32)],
        scratch_shapes=[pltpu.VMEM((R2, 128), jnp.float32)] * 3,
    )(pxg, pyg, pzg, pts_smem)


def _sa_call(XinT, W1, b1c, w1x, w1y, w1z, W2, b2c, W3, b3c,
             centers, cxc, cyc, czc, pxr, pyr, pzr, radius, CB):
    Din, N = XinT.shape
    C = cxc.shape[0]
    H3 = W3.shape[0]
    r32 = np.float32(radius)
    ninf = np.float32(-np.inf)

    def body(x_ref, w1_ref, b1_ref, w1x_ref, w1y_ref, w1z_ref,
             w2_ref, b2_ref, w3_ref, b3_ref, c_ref,
             cx_ref, cy_ref, cz_ref, px_ref, py_ref, pz_ref, o_ref):
        bf = jnp.bfloat16
        baseT = (jnp.dot(w1_ref[...], x_ref[...],
                         preferred_element_type=jnp.float32)
                 + b1_ref[...]).astype(bf)
        w1xv = w1x_ref[...]
        w1yv = w1y_ref[...]
        w1zv = w1z_ref[...]
        W2 = w2_ref[...].astype(bf)
        b2v = b2_ref[...].astype(bf)
        W3 = w3_ref[...].astype(bf)
        b3v = b3_ref[...]
        dx = cx_ref[...] - px_ref[...]
        dy = cy_ref[...] - py_ref[...]
        dz = cz_ref[...] - pz_ref[...]
        pen = jnp.where(
            jnp.sqrt(dx * dx + dy * dy + dz * dz) < r32, 0.0, ninf).astype(bf)
        li = lax.broadcasted_iota(jnp.int32, (H3, CB), 1)
        acc = jnp.zeros((H3, CB), jnp.float32)
        for c in range(CB):
            cxs = c_ref[c, 0]
            cys = c_ref[c, 1]
            czs = c_ref[c, 2]
            coffT = (cxs * w1xv + cys * w1yv + czs * w1zv).astype(bf)
            h = jnp.maximum(baseT - coffT, 0.0)
            h = jnp.maximum(
                jnp.dot(W2, h,
                        preferred_element_type=jnp.float32).astype(bf)
                + b2v, 0.0)
            h = jnp.dot(W3, h, preferred_element_type=jnp.float32).astype(bf)
            m = jnp.max(h + pen[c:c + 1, :], axis=1, keepdims=True)
            acc = jnp.where(li == c,
                            jnp.maximum(m.astype(jnp.float32) + b3v, 0.0),
                            acc)
        o_ref[0] = acc

    return pl.pallas_call(
        body,
        grid=(C // CB,),
        in_specs=[
            pl.BlockSpec((Din, N), lambda i: (0, 0)),
            pl.BlockSpec(W1.shape, lambda i: (0, 0)),
            pl.BlockSpec(b1c.shape, lambda i: (0, 0)),
            pl.BlockSpec(w1x.shape, lambda i: (0, 0)),
            pl.BlockSpec(w1y.shape, lambda i: (0, 0)),
            pl.BlockSpec(w1z.shape, lambda i: (0, 0)),
            pl.BlockSpec(W2.shape, lambda i: (0, 0)),
            pl.BlockSpec(b2c.shape, lambda i: (0, 0)),
            pl.BlockSpec(W3.shape, lambda i: (0, 0)),
            pl.BlockSpec(b3c.shape, lambda i: (0, 0)),
            pl.BlockSpec((CB, 3), lambda i: (i, 0),
                         memory_space=pltpu.SMEM),
            pl.BlockSpec((CB, 1), lambda i: (i, 0)),
            pl.BlockSpec((CB, 1), lambda i: (i, 0)),
            pl.BlockSpec((CB, 1), lambda i: (i, 0)),
            pl.BlockSpec((1, N), lambda i: (0, 0)),
            pl.BlockSpec((1, N), lambda i: (0, 0)),
            pl.BlockSpec((1, N), lambda i: (0, 0)),
        ],
        out_specs=pl.BlockSpec((1, H3, CB), lambda i: (i, 0, 0)),
        out_shape=jax.ShapeDtypeStruct((C // CB, H3, CB), jnp.float32),
    )(XinT, W1, b1c, w1x, w1y, w1z, W2, b2c, W3, b3c,
      centers, cxc, cyc, czc, pxr, pyr, pzr)


def _tail_call(c2cols, c2smem, f2, w1c, w1f, b1, w2, b2, w3, b3,
               wa, wb, bb, g0, be0, w2f, b2f, g1, be1):
    C2, F2 = f2.shape
    OUT = w2f.shape[1]

    def body(cx_ref, cy_ref, cz_ref, cs_ref, f_ref, w1c_ref, w1f_ref,
             b1_ref, w2_ref, b2_ref, w3_ref, b3_ref, wa_ref, wb_ref,
             bb_ref, g0_ref, be0_ref, w2f_ref, b2f_ref, g1_ref, be1_ref,
             o_ref):
        dx = cx_ref[...] - cs_ref[0, 0]
        dy = cy_ref[...] - cs_ref[0, 1]
        dz = cz_ref[...] - cs_ref[0, 2]
        W1c = w1c_ref[...]
        dpart = dx * W1c[0:1] + dy * W1c[1:2] + dz * W1c[2:3]
        f2v = f_ref[...]
        h = jnp.maximum(
            dpart
            + jnp.dot(f2v, w1f_ref[...], preferred_element_type=jnp.float32)
            + b1_ref[...], 0.0)
        h = jnp.maximum(
            jnp.dot(h, w2_ref[...], preferred_element_type=jnp.float32)
            + b2_ref[...], 0.0)
        h = jnp.maximum(
            jnp.dot(h, w3_ref[...], preferred_element_type=jnp.float32)
            + b3_ref[...], 0.0)
        fm = jnp.max(h, axis=0, keepdims=True)
        kmw = jnp.dot(fm, wb_ref[...], preferred_element_type=jnp.float32)
        y = (jnp.dot(f2v, wa_ref[...], preferred_element_type=jnp.float32)
             + kmw + bb_ref[...])
        y = jnp.maximum(g0_ref[...] * y * _INV_BN + be0_ref[...], 0.0)
        y = jnp.dot(y, w2f_ref[...], preferred_element_type=jnp.float32) \
            + b2f_ref[...]
        y = jnp.maximum(g1_ref[...] * y * _INV_BN + be1_ref[...], 0.0)
        o_ref[...] = y

    vspec = lambda a: pl.BlockSpec(a.shape, lambda: (0,) * a.ndim)
    args = (*c2cols, c2smem, f2, w1c, w1f, b1, w2, b2, w3, b3,
            wa, wb, bb, g0, be0, w2f, b2f, g1, be1)
    in_specs = [vspec(a) for a in args]
    in_specs[3] = pl.BlockSpec(c2smem.shape, lambda: (0, 0),
                               memory_space=pltpu.SMEM)
    return pl.pallas_call(
        body,
        in_specs=in_specs,
        out_specs=pl.BlockSpec((C2, OUT), lambda: (0, 0)),
        out_shape=jax.ShapeDtypeStruct((C2, OUT), jnp.float32),
    )(*args)


_NC, _NS, _L = 2, 16, 16
_NW = _NC * _NS


def _knn4_sc_pair(stage_a, stage_b):
    f32 = jnp.float32
    i32 = jnp.int32
    L = _L
    stages = []
    args = []
    out_type = []
    scratch = []
    for qcoords, pcoords in (stage_a, stage_b):
        Q = qcoords.shape[0]
        P = pcoords.shape[0]
        nq = Q // _NW
        stages.append((Q, P, nq, nq // L))
        args += [qcoords[:, 0], qcoords[:, 1], qcoords[:, 2],
                 jnp.repeat(pcoords[:, 0], L),
                 jnp.repeat(pcoords[:, 1], L),
                 jnp.repeat(pcoords[:, 2], L)]
        out_type += ([jax.ShapeDtypeStruct((Q,), f32) for _ in range(4)]
                     + [jax.ShapeDtypeStruct((Q,), i32) for _ in range(4)])
        scratch += ([pltpu.VMEM((nq,), f32) for _ in range(3)]
                    + [pltpu.VMEM((P * L,), f32) for _ in range(3)]
                    + [pltpu.VMEM((nq,), f32) for _ in range(4)]
                    + [pltpu.VMEM((nq,), i32) for _ in range(4)])

    mesh = plsc.VectorSubcoreMesh(core_axis_name="c", subcore_axis_name="s")

    @functools.partial(pl.kernel, mesh=mesh, out_type=out_type,
                       scratch_types=scratch)
    def knn_kernel(*refs):
        in_refs = refs[:12]
        out_refs = refs[12:28]
        scr_refs = refs[28:]
        wid = lax.axis_index("s") * _NC + lax.axis_index("c")
        for si, (Q, P, nq, ng) in enumerate(stages):
            qx_h, qy_h, qz_h, px_h, py_h, pz_h = in_refs[si * 6:si * 6 + 6]
            (sq0_h, sq1_h, sq2_h, sq3_h,
             ik0_h, ik1_h, ik2_h, ik3_h) = out_refs[si * 8:si * 8 + 8]
            (qxv, qyv, qzv, pxv, pyv, pzv,
             t0v, t1v, t2v, t3v, i0v, i1v, i2v, i3v) = \
                scr_refs[si * 14:si * 14 + 14]
            base = wid * nq
            pltpu.sync_copy(qx_h.at[pl.ds(base, nq)], qxv)
            pltpu.sync_copy(qy_h.at[pl.ds(base, nq)], qyv)
            pltpu.sync_copy(qz_h.at[pl.ds(base, nq)], qzv)
            pltpu.sync_copy(px_h, pxv)
            pltpu.sync_copy(py_h, pyv)
            pltpu.sync_copy(pz_h, pzv)
            for g in range(ng):
                qxg = qxv[pl.ds(g * L, L)]
                qyg = qyv[pl.ds(g * L, L)]
                qzg = qzv[pl.ds(g * L, L)]
                inf16 = jnp.full((L,), np.float32(np.inf), f32)
                zero16 = jnp.zeros((L,), i32)
                state = (inf16, inf16, inf16, inf16,
                         zero16, zero16, zero16, zero16)

                def point_body(p, st, qxg=qxg, qyg=qyg, qzg=qzg,
                               pxv=pxv, pyv=pyv, pzv=pzv):
                    t0, t1, t2, t3, i0, i1, i2, i3 = st
                    pxs = pxv[pl.ds(p * L, L)]
                    pys = pyv[pl.ds(p * L, L)]
                    pzs = pzv[pl.ds(p * L, L)]
                    dx = qxg - pxs
                    dy = qyg - pys
                    dz = qzg - pzs
                    dv = dx * dx + dy * dy + dz * dz
                    iv = jnp.full((L,), 0, i32) + p
                    c0 = dv < t0
                    t0n = jnp.where(c0, dv, t0)
                    i0n = jnp.where(c0, iv, i0)
                    dv1 = jnp.where(c0, t0, dv)
                    iv1 = jnp.where(c0, i0, iv)
                    c1 = dv1 < t1
                    t1n = jnp.where(c1, dv1, t1)
                    i1n = jnp.where(c1, iv1, i1)
                    dv2 = jnp.where(c1, t1, dv1)
                    iv2 = jnp.where(c1, i1, iv1)
                    c2 = dv2 < t2
                    t2n = jnp.where(c2, dv2, t2)
                    i2n = jnp.where(c2, iv2, i2)
                    dv3 = jnp.where(c2, t2, dv2)
                    iv3 = jnp.where(c2, i2, iv2)
                    c3 = dv3 < t3
                    t3n = jnp.where(c3, dv3, t3)
                    i3n = jnp.where(c3, iv3, i3)
                    return (t0n, t1n, t2n, t3n, i0n, i1n, i2n, i3n)

                st = lax.fori_loop(0, P, point_body, state, unroll=4)
                t0v[pl.ds(g * L, L)] = st[0]
                t1v[pl.ds(g * L, L)] = st[1]
                t2v[pl.ds(g * L, L)] = st[2]
                t3v[pl.ds(g * L, L)] = st[3]
                i0v[pl.ds(g * L, L)] = st[4]
                i1v[pl.ds(g * L, L)] = st[5]
                i2v[pl.ds(g * L, L)] = st[6]
                i3v[pl.ds(g * L, L)] = st[7]
            pltpu.sync_copy(t0v, sq0_h.at[pl.ds(base, nq)])
            pltpu.sync_copy(t1v, sq1_h.at[pl.ds(base, nq)])
            pltpu.sync_copy(t2v, sq2_h.at[pl.ds(base, nq)])
            pltpu.sync_copy(t3v, sq3_h.at[pl.ds(base, nq)])
            pltpu.sync_copy(i0v, ik0_h.at[pl.ds(base, nq)])
            pltpu.sync_copy(i1v, ik1_h.at[pl.ds(base, nq)])
            pltpu.sync_copy(i2v, ik2_h.at[pl.ds(base, nq)])
            pltpu.sync_copy(i3v, ik3_h.at[pl.ds(base, nq)])

    res = knn_kernel(*args)
    return ((list(res[0:4]), list(res[4:8])),
            (list(res[8:12]), list(res[12:16])))


def _fp_call(sqs, iks_in, from_f, f_prev, layer_arrays, bn_flags, QB):
    Q = sqs[0].shape[0]
    P, F = from_f.shape
    Dprev = f_prev.shape[1]
    OUT = layer_arrays[-1][0].shape[1]

    flat = []
    for arrs in layer_arrays:
        flat.extend(arrs)
    n_flat = len(flat)

    def body(*refs):
        s_refs = refs[0:4]
        i_refs = refs[4:8]
        ff_ref = refs[8]
        fp_ref = refs[9]
        lrefs = list(refs[10:10 + n_flat])
        o_ref = refs[10 + n_flat]
        cand = []
        for k in range(4):
            sq = s_refs[k][...]
            z = sq == 0.0
            d = jnp.where(z, 0.0, jnp.sqrt(jnp.where(z, 1.0, sq)))
            cand.append((d, i_refs[k][...]))

        def cswap(a, b):
            da, ia = a
            db, ib = b
            sw = (da > db) | ((da == db) & (ia > ib))
            lo = (jnp.where(sw, db, da), jnp.where(sw, ib, ia))
            hi = (jnp.where(sw, da, db), jnp.where(sw, ia, ib))
            return lo, hi

        for a, b in [(0, 1), (2, 3), (0, 2), (1, 3), (1, 2)]:
            cand[a], cand[b] = cswap(cand[a], cand[b])
        dks = [cand[k][0] for k in range(3)]
        iks = [cand[k][1] for k in range(3)]
        iz = [dk == 0.0 for dk in dks]
        any_zero = iz[0] | iz[1] | iz[2]
        raws = []
        for z, dk in zip(iz, dks):
            safe = jnp.where(z, 1.0, dk)
            raws.append(1.0 / (safe * safe))
        s = raws[0] + raws[1] + raws[2]
        col = lax.broadcasted_iota(jnp.int32, (QB, P), 1)
        Wc = jnp.zeros((QB, P), jnp.float32)
        for k in range(3):
            wk = jnp.where(any_zero, iz[k].astype(jnp.float32), raws[k] / s)
            Wc = Wc + jnp.where(col == iks[k], wk, 0.0)
        km = jnp.dot(Wc, ff_ref[...], preferred_element_type=jnp.float32)
        wp = lrefs[0][...]
        wk_ = lrefs[1][...]
        b = lrefs[2][...]
        x = (jnp.dot(fp_ref[...], wp, preferred_element_type=jnp.float32)
             + jnp.dot(km, wk_, preferred_element_type=jnp.float32) + b)
        li = 3
        if bn_flags[0]:
            x = jnp.maximum(lrefs[li][...] * x * _INV_BN
                            + lrefs[li + 1][...], 0.0)
            li += 2
        for has_bn in bn_flags[1:]:
            w = lrefs[li][...]
            b = lrefs[li + 1][...]
            x = jnp.dot(x, w, preferred_element_type=jnp.float32) + b
            li += 2
            if has_bn:
                x = jnp.maximum(lrefs[li][...] * x * _INV_BN
                                + lrefs[li + 1][...], 0.0)
                li += 2
        o_ref[...] = x

    def full2(a):
        s = a.shape
        return pl.BlockSpec(s, lambda i: (0, 0))

    qspec = pl.BlockSpec((QB, 1), lambda i: (i, 0))
    in_specs = ([qspec] * 8 + [
        full2(from_f),
        pl.BlockSpec((QB, Dprev), lambda i: (i, 0)),
    ] + [full2(a) for a in flat])
    return pl.pallas_call(
        body,
        grid=(Q // QB,),
        in_specs=in_specs,
        out_specs=pl.BlockSpec((QB, OUT), lambda i: (i, 0)),
        out_shape=jax.ShapeDtypeStruct((Q, OUT), jnp.float32),
    )(*sqs, *iks_in, from_f, f_prev, *flat)


def kernel(coords, features, params):
    coords = coords.astype(jnp.float32)
    features = features.astype(jnp.float32)
    N = coords.shape[0]

    pxg = coords[:, 0].reshape(N // 128, 128)
    pyg = coords[:, 1].reshape(N // 128, 128)
    pzg = coords[:, 2].reshape(N // 128, 128)

    c1flat, c2flat = _fps_pair_call(pxg, pyg, pzg, coords.reshape(1, -1),
                                    512, 128)
    c1coords = c1flat.reshape(512, 3)
    c2coords = c2flat.reshape(128, 3)
    c1x = c1coords[:, 0].reshape(4, 128)
    c1y = c1coords[:, 1].reshape(4, 128)
    c1z = c1coords[:, 2].reshape(4, 128)
    c2x = c2coords[:, 0].reshape(1, 128)
    c2y = c2coords[:, 1].reshape(1, 128)
    c2z = c2coords[:, 2].reshape(1, 128)

    (sq2s, ik2s), (sq1s, ik1s) = _knn4_sc_pair(
        (coords, c1coords), (c1coords, c2coords))

    def _unblock(o):
        return jnp.transpose(o, (1, 0, 2)).reshape(o.shape[1], -1)

    sa0 = params['sa'][0]
    XinT1 = jnp.concatenate([coords.T, features.T], axis=0)
    f1T = _sa_call(
        XinT1, sa0['W1'], sa0['b1'][:, None],
        sa0['W1'][:, 0:1], sa0['W1'][:, 1:2], sa0['W1'][:, 2:3],
        sa0['W2'], sa0['b2'][:, None], sa0['W3'], sa0['b3'][:, None],
        c1coords,
        c1x.reshape(512, 1), c1y.reshape(512, 1), c1z.reshape(512, 1),
        coords[:, 0].reshape(1, N), coords[:, 1].reshape(1, N),
        coords[:, 2].reshape(1, N),
        0.2, CB=16)
    f1T = _unblock(f1T)
    f1 = f1T.T

    sa1 = params['sa'][1]
    c1coordsT = jnp.stack(
        [c1x.reshape(-1), c1y.reshape(-1), c1z.reshape(-1)], axis=0)
    XinT2 = jnp.concatenate([c1coordsT, f1T], axis=0)
    f2T = _sa_call(
        XinT2, sa1['W1'], sa1['b1'][:, None],
        sa1['W1'][:, 0:1], sa1['W1'][:, 1:2], sa1['W1'][:, 2:3],
        sa1['W2'], sa1['b2'][:, None], sa1['W3'], sa1['b3'][:, None],
        c2coords,
        c2x.reshape(128, 1), c2y.reshape(128, 1), c2z.reshape(128, 1),
        c1x.reshape(1, 512), c1y.reshape(1, 512), c1z.reshape(1, 512),
        0.4, CB=16)
    f2 = _unblock(f2T).T

    sa2 = params['sa'][2]
    fp0 = params['fp'][0]
    W1t = sa2['W1'].T
    fp0W0t = fp0[0]['W'].T
    g2 = _tail_call(
        (c2x.reshape(128, 1), c2y.reshape(128, 1), c2z.reshape(128, 1)),
        c2coords, f2,
        W1t[:3], W1t[3:], sa2['b1'][None, :],
        sa2['W2'].T, sa2['b2'][None, :], sa2['W3'].T, sa2['b3'][None, :],
        fp0W0t[:256], fp0W0t[256:], fp0[0]['b'][None, :],
        fp0[0]['gamma'][None, :], fp0[0]['beta'][None, :],
        fp0[1]['W'].T, fp0[1]['b'][None, :],
        fp0[1]['gamma'][None, :], fp0[1]['beta'][None, :])

    fp1 = params['fp'][1]
    W0t = fp1[0]['W'].T
    layer_arrays1 = [
        (W0t[:128], W0t[128:], fp1[0]['b'][None, :],
         fp1[0]['gamma'][None, :], fp1[0]['beta'][None, :]),
        (fp1[1]['W'].T, fp1[1]['b'][None, :],
         fp1[1]['gamma'][None, :], fp1[1]['beta'][None, :]),
    ]
    g1 = _fp_call(
        [a.reshape(512, 1) for a in sq1s],
        [a.reshape(512, 1) for a in ik1s],
        g2, f1, layer_arrays1, [True, True], QB=512)

    fp2 = params['fp'][2]
    W0t2 = fp2[0]['W'].T
    layer_arrays2 = [
        (W0t2[:3], W0t2[3:], fp2[0]['b'][None, :],
         fp2[0]['gamma'][None, :], fp2[0]['beta'][None, :]),
        (fp2[1]['W'].T, fp2[1]['b'][None, :],
         fp2[1]['gamma'][None, :], fp2[1]['beta'][None, :]),
        (fp2[2]['W'].T, fp2[2]['b'][None, :]),
    ]
    out = _fp_call(
        [a.reshape(4096, 1) for a in sq2s],
        [a.reshape(4096, 1) for a in ik2s],
        g1, features, layer_arrays2, [True, True, False], QB=1024)
    return out


# speedup vs baseline: 1.2247x; 1.0154x over previous
import functools
import numpy as np
import jax
import jax.numpy as jnp
from jax import lax
from jax.experimental import pallas as pl
from jax.experimental.pallas import tpu as pltpu
from jax.experimental.pallas import tpu_sc as plsc

_INV_BN = np.float32(1.0) / np.sqrt(np.float32(1.0 + 1e-5))


def _flat_iota(shape):
    return (lax.broadcasted_iota(jnp.int32, shape, 0) * shape[1]
            + lax.broadcasted_iota(jnp.int32, shape, 1))


def _fps_pair_call(pxg, pyg, pzg, pts_smem, K1, K2):
    R = pxg.shape[0]
    R2 = K1 // 128

    def one_level(px, py, pz, shape, ps_read, o_ref, K, accum):
        fi = _flat_iota(shape)

        def dist_to(xj, yj, zj):
            dx = px - xj
            dy = py - yj
            dz = pz - zj
            return jnp.sqrt(dx * dx + dy * dy + dz * dz)

        x0, y0, z0 = ps_read(jnp.int32(0))
        o_ref[0, 0] = x0
        o_ref[0, 1] = y0
        o_ref[0, 2] = z0
        if accum is not None:
            accum(jnp.int32(0), x0, y0, z0)

        def step(i, d):
            j = jnp.argmax(d).astype(jnp.int32)
            xj, yj, zj = ps_read(j)
            o_ref[0, i * 3] = xj
            o_ref[0, i * 3 + 1] = yj
            o_ref[0, i * 3 + 2] = zj
            if accum is not None:
                accum(i, xj, yj, zj)
            return jnp.minimum(d, dist_to(xj, yj, zj))

        lax.fori_loop(1, K, step, dist_to(x0, y0, z0))

    def body(px_ref, py_ref, pz_ref, ps_ref, o1_ref, o2_ref,
             cx_s, cy_s, cz_s):
        ki = _flat_iota((R2, 128))

        def read1(j):
            return (ps_ref[0, j * 3], ps_ref[0, j * 3 + 1],
                    ps_ref[0, j * 3 + 2])

        def accum1(i, xj, yj, zj):
            sel = ki == i
            cx_s[...] = jnp.where(sel, xj, cx_s[...])
            cy_s[...] = jnp.where(sel, yj, cy_s[...])
            cz_s[...] = jnp.where(sel, zj, cz_s[...])

        one_level(px_ref[...], py_ref[...], pz_ref[...], (R, 128),
                  read1, o1_ref, K1, accum1)

        def read2(j):
            return (o1_ref[0, j * 3], o1_ref[0, j * 3 + 1],
                    o1_ref[0, j * 3 + 2])

        one_level(cx_s[...], cy_s[...], cz_s[...], (R2, 128),
                  read2, o2_ref, K2, None)

    return pl.pallas_call(
        body,
        in_specs=[
            pl.BlockSpec((R, 128), lambda: (0, 0)),
            pl.BlockSpec((R, 128), lambda: (0, 0)),
            pl.BlockSpec((R, 128), lambda: (0, 0)),
            pl.BlockSpec((1, 3 * R * 128), lambda: (0, 0),
                         memory_space=pltpu.SMEM),
        ],
        out_specs=[
            pl.BlockSpec((1, 3 * K1), lambda: (0, 0),
                         memory_space=pltpu.SMEM),
            pl.BlockSpec((1, 3 * K2), lambda: (0, 0),
                         memory_space=pltpu.SMEM),
        ],
        out_shape=[jax.ShapeDtypeStruct((1, 3 * K1), jnp.float32),
                   jax.ShapeDtypeStruct((1, 3 * K2), jnp.float32)],
        scratch_shapes=[pltpu.VMEM((R2, 128), jnp.float32)] * 3,
    )(pxg, pyg, pzg, pts_smem)


def _sa_call(XinT, W1, b1c, w1x, w1y, w1z, W2, b2c, W3, b3c,
             centers, cxc, cyc, czc, pxr, pyr, pzr, radius, CB):
    Din, N = XinT.shape
    C = cxc.shape[0]
    H3 = W3.shape[0]
    r32 = np.float32(radius)
    ninf = np.float32(-np.inf)

    def body(x_ref, w1_ref, b1_ref, w1x_ref, w1y_ref, w1z_ref,
             w2_ref, b2_ref, w3_ref, b3_ref, c_ref,
             cx_ref, cy_ref, cz_ref, px_ref, py_ref, pz_ref, o_ref):
        bf = jnp.bfloat16
        baseT = (jnp.dot(w1_ref[...], x_ref[...],
                         preferred_element_type=jnp.float32)
                 + b1_ref[...]).astype(bf)
        w1xv = w1x_ref[...]
        w1yv = w1y_ref[...]
        w1zv = w1z_ref[...]
        W2 = w2_ref[...].astype(bf)
        b2v = b2_ref[...].astype(bf)
        W3 = w3_ref[...].astype(bf)
        b3v = b3_ref[...]
        dx = cx_ref[...] - px_ref[...]
        dy = cy_ref[...] - py_ref[...]
        dz = cz_ref[...] - pz_ref[...]
        pen = jnp.where(
            jnp.sqrt(dx * dx + dy * dy + dz * dz) < r32, 0.0, ninf).astype(bf)
        li = lax.broadcasted_iota(jnp.int32, (H3, CB), 1)
        acc = jnp.zeros((H3, CB), jnp.float32)
        for c in range(CB):
            cxs = c_ref[c, 0]
            cys = c_ref[c, 1]
            czs = c_ref[c, 2]
            coffT = (cxs * w1xv + cys * w1yv + czs * w1zv).astype(bf)
            h = jnp.maximum(baseT - coffT, 0.0)
            h = jnp.maximum(
                jnp.dot(W2, h,
                        preferred_element_type=jnp.float32).astype(bf)
                + b2v, 0.0)
            h = jnp.dot(W3, h, preferred_element_type=jnp.float32).astype(bf)
            m = jnp.max(h + pen[c:c + 1, :], axis=1, keepdims=True)
            acc = jnp.where(li == c,
                            jnp.maximum(m.astype(jnp.float32) + b3v, 0.0),
                            acc)
        o_ref[0] = acc

    return pl.pallas_call(
        body,
        grid=(C // CB,),
        in_specs=[
            pl.BlockSpec((Din, N), lambda i: (0, 0)),
            pl.BlockSpec(W1.shape, lambda i: (0, 0)),
            pl.BlockSpec(b1c.shape, lambda i: (0, 0)),
            pl.BlockSpec(w1x.shape, lambda i: (0, 0)),
            pl.BlockSpec(w1y.shape, lambda i: (0, 0)),
            pl.BlockSpec(w1z.shape, lambda i: (0, 0)),
            pl.BlockSpec(W2.shape, lambda i: (0, 0)),
            pl.BlockSpec(b2c.shape, lambda i: (0, 0)),
            pl.BlockSpec(W3.shape, lambda i: (0, 0)),
            pl.BlockSpec(b3c.shape, lambda i: (0, 0)),
            pl.BlockSpec((CB, 3), lambda i: (i, 0),
                         memory_space=pltpu.SMEM),
            pl.BlockSpec((CB, 1), lambda i: (i, 0)),
            pl.BlockSpec((CB, 1), lambda i: (i, 0)),
            pl.BlockSpec((CB, 1), lambda i: (i, 0)),
            pl.BlockSpec((1, N), lambda i: (0, 0)),
            pl.BlockSpec((1, N), lambda i: (0, 0)),
            pl.BlockSpec((1, N), lambda i: (0, 0)),
        ],
        out_specs=pl.BlockSpec((1, H3, CB), lambda i: (i, 0, 0)),
        out_shape=jax.ShapeDtypeStruct((C // CB, H3, CB), jnp.float32),
    )(XinT, W1, b1c, w1x, w1y, w1z, W2, b2c, W3, b3c,
      centers, cxc, cyc, czc, pxr, pyr, pzr)


def _tail_call(c2cols, c2smem, f2, w1c, w1f, b1, w2, b2, w3, b3,
               wa, wb, bb, g0, be0, w2f, b2f, g1, be1):
    C2, F2 = f2.shape
    OUT = w2f.shape[1]

    def body(cx_ref, cy_ref, cz_ref, cs_ref, f_ref, w1c_ref, w1f_ref,
             b1_ref, w2_ref, b2_ref, w3_ref, b3_ref, wa_ref, wb_ref,
             bb_ref, g0_ref, be0_ref, w2f_ref, b2f_ref, g1_ref, be1_ref,
             o_ref):
        dx = cx_ref[...] - cs_ref[0, 0]
        dy = cy_ref[...] - cs_ref[0, 1]
        dz = cz_ref[...] - cs_ref[0, 2]
        W1c = w1c_ref[...]
        dpart = dx * W1c[0:1] + dy * W1c[1:2] + dz * W1c[2:3]
        f2v = f_ref[...]
        h = jnp.maximum(
            dpart
            + jnp.dot(f2v, w1f_ref[...], preferred_element_type=jnp.float32)
            + b1_ref[...], 0.0)
        h = jnp.maximum(
            jnp.dot(h, w2_ref[...], preferred_element_type=jnp.float32)
            + b2_ref[...], 0.0)
        h = jnp.maximum(
            jnp.dot(h, w3_ref[...], preferred_element_type=jnp.float32)
            + b3_ref[...], 0.0)
        fm = jnp.max(h, axis=0, keepdims=True)
        kmw = jnp.dot(fm, wb_ref[...], preferred_element_type=jnp.float32)
        y = (jnp.dot(f2v, wa_ref[...], preferred_element_type=jnp.float32)
             + kmw + bb_ref[...])
        y = jnp.maximum(g0_ref[...] * y * _INV_BN + be0_ref[...], 0.0)
        y = jnp.dot(y, w2f_ref[...], preferred_element_type=jnp.float32) \
            + b2f_ref[...]
        y = jnp.maximum(g1_ref[...] * y * _INV_BN + be1_ref[...], 0.0)
        o_ref[...] = y

    vspec = lambda a: pl.BlockSpec(a.shape, lambda: (0,) * a.ndim)
    args = (*c2cols, c2smem, f2, w1c, w1f, b1, w2, b2, w3, b3,
            wa, wb, bb, g0, be0, w2f, b2f, g1, be1)
    in_specs = [vspec(a) for a in args]
    in_specs[3] = pl.BlockSpec(c2smem.shape, lambda: (0, 0),
                               memory_space=pltpu.SMEM)
    return pl.pallas_call(
        body,
        in_specs=in_specs,
        out_specs=pl.BlockSpec((C2, OUT), lambda: (0, 0)),
        out_shape=jax.ShapeDtypeStruct((C2, OUT), jnp.float32),
    )(*args)


_NC, _NS, _L = 2, 16, 16
_NW = _NC * _NS


def _knn4_sc_pair(stage_a, stage_b):
    f32 = jnp.float32
    i32 = jnp.int32
    L = _L
    stages = []
    args = []
    out_type = []
    scratch = []
    for qcoords, pcoords in (stage_a, stage_b):
        Q = qcoords.shape[0]
        P = pcoords.shape[0]
        nq = Q // _NW
        stages.append((Q, P, nq, nq // L))
        args += [qcoords[:, 0], qcoords[:, 1], qcoords[:, 2],
                 jnp.repeat(pcoords[:, 0], L),
                 jnp.repeat(pcoords[:, 1], L),
                 jnp.repeat(pcoords[:, 2], L)]
        out_type += ([jax.ShapeDtypeStruct((Q,), f32) for _ in range(4)]
                     + [jax.ShapeDtypeStruct((Q,), i32) for _ in range(4)])
        scratch += ([pltpu.VMEM((nq,), f32) for _ in range(3)]
                    + [pltpu.VMEM((P * L,), f32) for _ in range(3)]
                    + [pltpu.VMEM((nq,), f32) for _ in range(4)]
                    + [pltpu.VMEM((nq,), i32) for _ in range(4)])

    mesh = plsc.VectorSubcoreMesh(core_axis_name="c", subcore_axis_name="s")

    @functools.partial(pl.kernel, mesh=mesh, out_type=out_type,
                       scratch_types=scratch)
    def knn_kernel(*refs):
        in_refs = refs[:12]
        out_refs = refs[12:28]
        scr_refs = refs[28:]
        wid = lax.axis_index("s") * _NC + lax.axis_index("c")
        for si, (Q, P, nq, ng) in enumerate(stages):
            qx_h, qy_h, qz_h, px_h, py_h, pz_h = in_refs[si * 6:si * 6 + 6]
            (sq0_h, sq1_h, sq2_h, sq3_h,
             ik0_h, ik1_h, ik2_h, ik3_h) = out_refs[si * 8:si * 8 + 8]
            (qxv, qyv, qzv, pxv, pyv, pzv,
             t0v, t1v, t2v, t3v, i0v, i1v, i2v, i3v) = \
                scr_refs[si * 14:si * 14 + 14]
            base = wid * nq
            pltpu.sync_copy(qx_h.at[pl.ds(base, nq)], qxv)
            pltpu.sync_copy(qy_h.at[pl.ds(base, nq)], qyv)
            pltpu.sync_copy(qz_h.at[pl.ds(base, nq)], qzv)
            pltpu.sync_copy(px_h, pxv)
            pltpu.sync_copy(py_h, pyv)
            pltpu.sync_copy(pz_h, pzv)
            for g in range(ng):
                qxg = qxv[pl.ds(g * L, L)]
                qyg = qyv[pl.ds(g * L, L)]
                qzg = qzv[pl.ds(g * L, L)]
                inf16 = jnp.full((L,), np.float32(np.inf), f32)
                zero16 = jnp.zeros((L,), i32)
                state = (inf16, inf16, inf16, inf16,
                         zero16, zero16, zero16, zero16)

                def point_body(p, st, qxg=qxg, qyg=qyg, qzg=qzg,
                               pxv=pxv, pyv=pyv, pzv=pzv):
                    t0, t1, t2, t3, i0, i1, i2, i3 = st
                    pxs = pxv[pl.ds(p * L, L)]
                    pys = pyv[pl.ds(p * L, L)]
                    pzs = pzv[pl.ds(p * L, L)]
                    dx = qxg - pxs
                    dy = qyg - pys
                    dz = qzg - pzs
                    dv = dx * dx + dy * dy + dz * dz
                    iv = jnp.full((L,), 0, i32) + p
                    c0 = dv < t0
                    t0n = jnp.where(c0, dv, t0)
                    i0n = jnp.where(c0, iv, i0)
                    dv1 = jnp.where(c0, t0, dv)
                    iv1 = jnp.where(c0, i0, iv)
                    c1 = dv1 < t1
                    t1n = jnp.where(c1, dv1, t1)
                    i1n = jnp.where(c1, iv1, i1)
                    dv2 = jnp.where(c1, t1, dv1)
                    iv2 = jnp.where(c1, i1, iv1)
                    c2 = dv2 < t2
                    t2n = jnp.where(c2, dv2, t2)
                    i2n = jnp.where(c2, iv2, i2)
                    dv3 = jnp.where(c2, t2, dv2)
                    iv3 = jnp.where(c2, i2, iv2)
                    c3 = dv3 < t3
                    t3n = jnp.where(c3, dv3, t3)
                    i3n = jnp.where(c3, iv3, i3)
                    return (t0n, t1n, t2n, t3n, i0n, i1n, i2n, i3n)

                st = lax.fori_loop(0, P, point_body, state, unroll=4)
                t0v[pl.ds(g * L, L)] = st[0]
                t1v[pl.ds(g * L, L)] = st[1]
                t2v[pl.ds(g * L, L)] = st[2]
                t3v[pl.ds(g * L, L)] = st[3]
                i0v[pl.ds(g * L, L)] = st[4]
                i1v[pl.ds(g * L, L)] = st[5]
                i2v[pl.ds(g * L, L)] = st[6]
                i3v[pl.ds(g * L, L)] = st[7]
            pltpu.sync_copy(t0v, sq0_h.at[pl.ds(base, nq)])
            pltpu.sync_copy(t1v, sq1_h.at[pl.ds(base, nq)])
            pltpu.sync_copy(t2v, sq2_h.at[pl.ds(base, nq)])
            pltpu.sync_copy(t3v, sq3_h.at[pl.ds(base, nq)])
            pltpu.sync_copy(i0v, ik0_h.at[pl.ds(base, nq)])
            pltpu.sync_copy(i1v, ik1_h.at[pl.ds(base, nq)])
            pltpu.sync_copy(i2v, ik2_h.at[pl.ds(base, nq)])
            pltpu.sync_copy(i3v, ik3_h.at[pl.ds(base, nq)])

    res = knn_kernel(*args)
    return ((list(res[0:4]), list(res[4:8])),
            (list(res[8:12]), list(res[12:16])))


def _fp_call(sqs, iks_in, from_f, f_prev, layer_arrays, bn_flags, QB):
    Q = sqs[0].shape[0]
    P, F = from_f.shape
    Dprev = f_prev.shape[1]
    OUT = layer_arrays[-1][0].shape[1]

    flat = []
    for arrs in layer_arrays:
        flat.extend(arrs)
    n_flat = len(flat)

    def body(*refs):
        s_refs = refs[0:4]
        i_refs = refs[4:8]
        ff_ref = refs[8]
        fp_ref = refs[9]
        lrefs = list(refs[10:10 + n_flat])
        o_ref = refs[10 + n_flat]
        cand = []
        for k in range(4):
            sq = s_refs[k][...]
            z = sq == 0.0
            d = jnp.where(z, 0.0, jnp.sqrt(jnp.where(z, 1.0, sq)))
            cand.append((d, i_refs[k][...]))

        def cswap(a, b):
            da, ia = a
            db, ib = b
            sw = (da > db) | ((da == db) & (ia > ib))
            lo = (jnp.where(sw, db, da), jnp.where(sw, ib, ia))
            hi = (jnp.where(sw, da, db), jnp.where(sw, ia, ib))
            return lo, hi

        for a, b in [(0, 1), (2, 3), (0, 2), (1, 3), (1, 2)]:
            cand[a], cand[b] = cswap(cand[a], cand[b])
        dks = [cand[k][0] for k in range(3)]
        iks = [cand[k][1] for k in range(3)]
        iz = [dk == 0.0 for dk in dks]
        any_zero = iz[0] | iz[1] | iz[2]
        raws = []
        for z, dk in zip(iz, dks):
            safe = jnp.where(z, 1.0, dk)
            raws.append(1.0 / (safe * safe))
        s = raws[0] + raws[1] + raws[2]
        col = lax.broadcasted_iota(jnp.int32, (QB, P), 1)
        Wc = jnp.zeros((QB, P), jnp.float32)
        for k in range(3):
            wk = jnp.where(any_zero, iz[k].astype(jnp.float32), raws[k] / s)
            Wc = Wc + jnp.where(col == iks[k], wk, 0.0)
        km = jnp.dot(Wc, ff_ref[...], preferred_element_type=jnp.float32)
        wp = lrefs[0][...]
        wk_ = lrefs[1][...]
        b = lrefs[2][...]
        x = (jnp.dot(fp_ref[...], wp, preferred_element_type=jnp.float32)
             + jnp.dot(km, wk_, preferred_element_type=jnp.float32) + b)
        li = 3
        if bn_flags[0]:
            x = jnp.maximum(lrefs[li][...] * x * _INV_BN
                            + lrefs[li + 1][...], 0.0)
            li += 2
        for has_bn in bn_flags[1:]:
            w = lrefs[li][...]
            b = lrefs[li + 1][...]
            x = jnp.dot(x, w, preferred_element_type=jnp.float32) + b
            li += 2
            if has_bn:
                x = jnp.maximum(lrefs[li][...] * x * _INV_BN
                                + lrefs[li + 1][...], 0.0)
                li += 2
        o_ref[...] = x

    def full2(a):
        s = a.shape
        return pl.BlockSpec(s, lambda i: (0, 0))

    qspec = pl.BlockSpec((QB, 1), lambda i: (i, 0))
    in_specs = ([qspec] * 8 + [
        full2(from_f),
        pl.BlockSpec((QB, Dprev), lambda i: (i, 0)),
    ] + [full2(a) for a in flat])
    return pl.pallas_call(
        body,
        grid=(Q // QB,),
        in_specs=in_specs,
        out_specs=pl.BlockSpec((QB, OUT), lambda i: (i, 0)),
        out_shape=jax.ShapeDtypeStruct((Q, OUT), jnp.float32),
    )(*sqs, *iks_in, from_f, f_prev, *flat)


def kernel(coords, features, params):
    coords = coords.astype(jnp.float32)
    features = features.astype(jnp.float32)
    N = coords.shape[0]

    pxg = coords[:, 0].reshape(N // 128, 128)
    pyg = coords[:, 1].reshape(N // 128, 128)
    pzg = coords[:, 2].reshape(N // 128, 128)

    c1flat, c2flat = _fps_pair_call(pxg, pyg, pzg, coords.reshape(1, -1),
                                    512, 128)
    c1coords = c1flat.reshape(512, 3)
    c2coords = c2flat.reshape(128, 3)
    c1x = c1coords[:, 0].reshape(4, 128)
    c1y = c1coords[:, 1].reshape(4, 128)
    c1z = c1coords[:, 2].reshape(4, 128)
    c2x = c2coords[:, 0].reshape(1, 128)
    c2y = c2coords[:, 1].reshape(1, 128)
    c2z = c2coords[:, 2].reshape(1, 128)

    (sq2s, ik2s), (sq1s, ik1s) = _knn4_sc_pair(
        (coords, c1coords), (c1coords, c2coords))

    def _unblock(o):
        return jnp.transpose(o, (1, 0, 2)).reshape(o.shape[1], -1)

    sa0 = params['sa'][0]
    XinT1 = jnp.concatenate([coords.T, features.T], axis=0)
    f1T = _sa_call(
        XinT1, sa0['W1'], sa0['b1'][:, None],
        sa0['W1'][:, 0:1], sa0['W1'][:, 1:2], sa0['W1'][:, 2:3],
        sa0['W2'], sa0['b2'][:, None], sa0['W3'], sa0['b3'][:, None],
        c1coords,
        c1x.reshape(512, 1), c1y.reshape(512, 1), c1z.reshape(512, 1),
        coords[:, 0].reshape(1, N), coords[:, 1].reshape(1, N),
        coords[:, 2].reshape(1, N),
        0.2, CB=32)
    f1T = _unblock(f1T)
    f1 = f1T.T

    sa1 = params['sa'][1]
    c1coordsT = jnp.stack(
        [c1x.reshape(-1), c1y.reshape(-1), c1z.reshape(-1)], axis=0)
    XinT2 = jnp.concatenate([c1coordsT, f1T], axis=0)
    f2T = _sa_call(
        XinT2, sa1['W1'], sa1['b1'][:, None],
        sa1['W1'][:, 0:1], sa1['W1'][:, 1:2], sa1['W1'][:, 2:3],
        sa1['W2'], sa1['b2'][:, None], sa1['W3'], sa1['b3'][:, None],
        c2coords,
        c2x.reshape(128, 1), c2y.reshape(128, 1), c2z.reshape(128, 1),
        c1x.reshape(1, 512), c1y.reshape(1, 512), c1z.reshape(1, 512),
        0.4, CB=32)
    f2 = _unblock(f2T).T

    sa2 = params['sa'][2]
    fp0 = params['fp'][0]
    W1t = sa2['W1'].T
    fp0W0t = fp0[0]['W'].T
    g2 = _tail_call(
        (c2x.reshape(128, 1), c2y.reshape(128, 1), c2z.reshape(128, 1)),
        c2coords, f2,
        W1t[:3], W1t[3:], sa2['b1'][None, :],
        sa2['W2'].T, sa2['b2'][None, :], sa2['W3'].T, sa2['b3'][None, :],
        fp0W0t[:256], fp0W0t[256:], fp0[0]['b'][None, :],
        fp0[0]['gamma'][None, :], fp0[0]['beta'][None, :],
        fp0[1]['W'].T, fp0[1]['b'][None, :],
        fp0[1]['gamma'][None, :], fp0[1]['beta'][None, :])

    fp1 = params['fp'][1]
    W0t = fp1[0]['W'].T
    layer_arrays1 = [
        (W0t[:128], W0t[128:], fp1[0]['b'][None, :],
         fp1[0]['gamma'][None, :], fp1[0]['beta'][None, :]),
        (fp1[1]['W'].T, fp1[1]['b'][None, :],
         fp1[1]['gamma'][None, :], fp1[1]['beta'][None, :]),
    ]
    g1 = _fp_call(
        [a.reshape(512, 1) for a in sq1s],
        [a.reshape(512, 1) for a in ik1s],
        g2, f1, layer_arrays1, [True, True], QB=512)

    fp2 = params['fp'][2]
    W0t2 = fp2[0]['W'].T
    layer_arrays2 = [
        (W0t2[:3], W0t2[3:], fp2[0]['b'][None, :],
         fp2[0]['gamma'][None, :], fp2[0]['beta'][None, :]),
        (fp2[1]['W'].T, fp2[1]['b'][None, :],
         fp2[1]['gamma'][None, :], fp2[1]['beta'][None, :]),
        (fp2[2]['W'].T, fp2[2]['b'][None, :]),
    ]
    out = _fp_call(
        [a.reshape(4096, 1) for a in sq2s],
        [a.reshape(4096, 1) for a in ik2s],
        g1, features, layer_arrays2, [True, True, False], QB=1024)
    return out
